# baseline (speedup 1.0000x reference)
import functools
import jax
import jax.numpy as jnp
from jax import lax
from jax.experimental import pallas as pl
from jax.experimental.pallas import tpu as pltpu
from jax.experimental.pallas import tpu_sc as plsc

_NC = 2
_NS = 16
_NW = _NC * _NS


def _mesh():
    return plsc.VectorSubcoreMesh(core_axis_name="c", subcore_axis_name="s")


@functools.partial(jax.jit, static_argnames=("weighted",))
def _sc_rows_agg(h, Sidx, Tidx, w, *, weighted):
    n, D = h.shape
    E = Sidx.shape[0]
    per_w = E // _NW
    C = 80
    assert E % _NW == 0 and per_w % C == 0 and n % _NS == 0
    nchunks = per_w // C
    assert n % 16 == 0
    nblk = n // 16
    blk_iters = (nblk + _NS - 1) // _NS
    nD8 = D // 16

    def body(h_hbm, s_hbm, t_hbm, w_hbm, out_hbm, sidx_v, tidx_v, w_v,
             rows_v, zbuf_v, accum_sh, sem):
        cid = lax.axis_index("c")
        sid = lax.axis_index("s")
        wid = cid * _NS + sid
        base_w = wid * per_w

        zero16 = jnp.zeros((16,), jnp.float32)
        for r in range(16):
            for j in range(nD8):
                zbuf_v[r, pl.ds(j * 16, 16)] = zero16

        def zcopy(i, _):
            b = sid + i * _NS

            @pl.when(b < nblk)
            def _do():
                pltpu.sync_copy(zbuf_v, accum_sh.at[pl.ds(b * 16, 16)])

            return _

        lax.fori_loop(0, blk_iters, zcopy, 0)
        plsc.subcore_barrier()

        def chunk(i, _):
            base = base_w + i * C
            pltpu.sync_copy(s_hbm.at[pl.ds(base, C)], sidx_v)
            pltpu.sync_copy(t_hbm.at[pl.ds(base, C)], tidx_v)
            pltpu.async_copy(h_hbm.at[sidx_v], rows_v, sem).wait()
            if weighted:
                pltpu.sync_copy(w_hbm.at[pl.ds(base, C)], w_v)

                def mul_blk(b, _):
                    w16 = w_v[pl.ds(b * 16, 16)]
                    for l in range(16):
                        ws = w16[l]
                        r = b * 16 + l
                        for j in range(nD8):
                            sl = pl.ds(j * 16, 16)
                            rows_v[r, sl] = rows_v[r, sl] * ws
                    return _

                lax.fori_loop(0, C // 16, mul_blk, 0)
            pltpu.sync_copy(rows_v, accum_sh.at[tidx_v], add=True)
            return _

        lax.fori_loop(0, nchunks, chunk, 0)
        plsc.subcore_barrier()

        def wb(i, _):
            b = sid + i * _NS

            @pl.when(b < nblk)
            def _do():
                sl = pl.ds(b * 16, 16)
                pltpu.sync_copy(accum_sh.at[sl], zbuf_v)
                pltpu.sync_copy(zbuf_v, out_hbm.at[cid].at[sl])

            return _

        lax.fori_loop(0, blk_iters, wb, 0)

    f = pl.kernel(
        body,
        out_type=jax.ShapeDtypeStruct((_NC, n, D), jnp.float32),
        mesh=_mesh(),
        scratch_types=[
            pltpu.VMEM((C,), jnp.int32),
            pltpu.VMEM((C,), jnp.int32),
            pltpu.VMEM((C,), jnp.float32),
            pltpu.VMEM((C, D), jnp.float32),
            pltpu.VMEM((16, D), jnp.float32),
            pltpu.VMEM_SHARED((n, D), jnp.float32),
            pltpu.SemaphoreType.DMA,
        ],
    )
    return f(h, Sidx, Tidx, w)


@jax.jit
def _sc_count(Tidx, n):
    E = Tidx.shape[0]
    per_w = E // _NW
    C = 128
    assert E % _NW == 0 and per_w % C == 0 and n % (_NS * 8) == 0
    nchunks = per_w // C
    rows_per_sub = n // _NS

    def body(t_hbm, out_hbm, tidx_v, ones_v, zbuf_v, accum_sh):
        cid = lax.axis_index("c")
        sid = lax.axis_index("s")
        wid = cid * _NS + sid
        base_w = wid * per_w

        zero16 = jnp.zeros((16,), jnp.float32)
        one16 = jnp.ones((16,), jnp.float32)
        for j in range(C // 16):
            ones_v[pl.ds(j * 16, 16)] = one16

        def zrow(r, _):
            zbuf_v[pl.ds(r * 16, 16)] = zero16
            return _

        lax.fori_loop(0, rows_per_sub // 16, zrow, 0)
        pltpu.sync_copy(zbuf_v, accum_sh.at[pl.ds(sid * rows_per_sub,
                                                  rows_per_sub)])
        plsc.subcore_barrier()

        def chunk(i, _):
            base = base_w + i * C
            pltpu.sync_copy(t_hbm.at[pl.ds(base, C)], tidx_v)
            pltpu.sync_copy(ones_v, accum_sh.at[tidx_v], add=True)
            return _

        lax.fori_loop(0, nchunks, chunk, 0)
        plsc.subcore_barrier()

        sl = pl.ds(sid * rows_per_sub, rows_per_sub)
        pltpu.sync_copy(accum_sh.at[sl], zbuf_v)
        pltpu.sync_copy(zbuf_v, out_hbm.at[cid].at[sl])

    f = pl.kernel(
        body,
        out_type=jax.ShapeDtypeStruct((_NC, n), jnp.float32),
        mesh=_mesh(),
        scratch_types=[
            pltpu.VMEM((C,), jnp.int32),
            pltpu.VMEM((C,), jnp.float32),
            pltpu.VMEM((n // _NS,), jnp.float32),
            pltpu.VMEM_SHARED((n,), jnp.float32),
        ],
    )
    return f(Tidx)


def _seg_sum_rows(h, Sidx, Tidx, w, n):
    weighted = w is not None
    if not weighted:
        w = jnp.zeros((1,), h.dtype)
        w = jnp.broadcast_to(w, (Sidx.shape[0],))
    p = _sc_rows_agg(h, Sidx, Tidx, w, weighted=weighted)
    return p[0] + p[1]


def _seg_sum_scalar(vals, Tidx, n):
    return jnp.zeros((n,), vals.dtype).at[Tidx].add(vals)


def kernel(x, edge_index, batch, W1, b1, W2, b2, W3, b3, att1, att2,
           Wl1, bl1, Wl2, bl2, Wl3, bl3):
    n = x.shape[0]
    e = edge_index.shape[1]
    k1 = n // 2
    k2 = k1 // 2
    src = edge_index[0]
    dst = edge_index[1]
    f32 = x.dtype
    ones_e = jnp.ones((e,), f32)

    deg0 = _seg_sum_scalar(ones_e, dst, n) + 1.0
    dinv = jax.lax.rsqrt(jnp.maximum(deg0, 1.0))
    g = dinv[:, None] * (x @ W1 + b1)
    aggA = _seg_sum_rows(g, src, dst, None, n)
    h0 = jax.nn.relu(dinv[:, None] * (aggA + g))

    def pool(h, S, T, ew, k, sel_prev, unit_w=False):
        deg = _seg_sum_scalar(ew, T, n)
        agg = _seg_sum_rows(h, S, T, None if unit_w else ew, n) \
            / jnp.maximum(deg, 1e-9)[:, None]
        score = jnp.abs(h - agg).sum(-1)
        msc = score if sel_prev is None else jnp.where(sel_prev, score, -1.0)
        tau = jnp.sort(msc)[n - k]
        sel = msc >= tau
        n0 = jnp.argmax(msc)
        xn = jnp.where(sel[:, None], h * jnp.tanh(score)[:, None], 0.0)
        return sel, n0, xn

    def attention(xn, S, T, ew, sel, n0, att):
        a = xn @ att[:xn.shape[1]]
        b = xn @ att[xn.shape[1]:]
        valid = sel[S] & sel[T]
        logits = jax.nn.leaky_relu(a[S] + b[T], 0.2) + ew
        gm = jnp.max(jnp.where(valid, logits, -jnp.inf))
        ex = jnp.where(valid, jnp.exp(logits - gm), 0.0)
        den = _seg_sum_scalar(ex, T, n)
        new_ew = jnp.where(valid, ex / jnp.maximum(den[T], 1e-16), 0.0)
        return jnp.where(valid, S, n0), jnp.where(valid, T, n0), new_ew

    def readout(xn, k):
        return jnp.concatenate([jnp.max(xn, axis=0), jnp.sum(xn, axis=0) / k])[None, :]

    def gcn_w(xin, S, T, ew, W, b):
        h = xin @ W + b
        deg = _seg_sum_scalar(ew, T, n) + 1.0
        agg = _seg_sum_rows(h, S, T, ew, n) + h
        return jax.nn.relu(agg / deg[:, None])

    sel1, n01, xn1 = pool(h0, src, dst, ones_e, k1, None, unit_w=True)
    S1, T1, ew1 = attention(xn1, src, dst, ones_e, sel1, n01, att1)
    x1 = readout(xn1, k1)

    h1 = gcn_w(xn1, S1, T1, ew1, W2, b2)

    sel2, n02, xn2 = pool(h1, S1, T1, ew1, k2, sel1)
    S2, T2, ew2 = attention(xn2, S1, T1, ew1, sel2, n02, att2)
    x2 = readout(xn2, k2)

    h2 = jnp.where(sel2[:, None], gcn_w(xn2, S2, T2, ew2, W3, b3), 0.0)
    x3 = readout(h2, k2)

    xr = jax.nn.relu(x1) + jax.nn.relu(x2) + jax.nn.relu(x3)
    xr = jax.nn.relu(xr @ Wl1 + bl1)
    xr = jax.nn.relu(xr @ Wl2 + bl2)
    return jax.nn.log_softmax(xr @ Wl3 + bl3, axis=-1)


# speedup vs baseline: 1.0001x; 1.0001x over previous
import functools
import jax
import jax.numpy as jnp
from jax import lax
from jax.experimental import pallas as pl
from jax.experimental.pallas import tpu as pltpu
from jax.experimental.pallas import tpu_sc as plsc

_NC = 2
_NS = 16
_NW = _NC * _NS


def _mesh():
    return plsc.VectorSubcoreMesh(core_axis_name="c", subcore_axis_name="s")


@functools.partial(jax.jit, static_argnames=("weighted",))
def _sc_rows_agg(h, Sidx, Tidx, w, *, weighted):
    n, D = h.shape
    E = Sidx.shape[0]
    per_w = E // _NW
    C = 80
    assert E % _NW == 0 and per_w % C == 0 and n % _NS == 0
    nchunks = per_w // C
    assert n % 16 == 0
    nblk = n // 16
    blk_iters = (nblk + _NS - 1) // _NS
    nD8 = D // 16

    def body(h_hbm, s_hbm, t_hbm, w_hbm, out_hbm, sidx_v, tidx_v, w_v,
             rows_v, zbuf_v, accum_sh, sem):
        cid = lax.axis_index("c")
        sid = lax.axis_index("s")
        wid = cid * _NS + sid
        base_w = wid * per_w

        zero16 = jnp.zeros((16,), jnp.float32)
        for r in range(16):
            for j in range(nD8):
                zbuf_v[r, pl.ds(j * 16, 16)] = zero16

        def zcopy(i, _):
            b = sid + i * _NS

            @pl.when(b < nblk)
            def _do():
                pltpu.sync_copy(zbuf_v, accum_sh.at[pl.ds(b * 16, 16)])

            return _

        lax.fori_loop(0, blk_iters, zcopy, 0)
        plsc.subcore_barrier()

        def chunk(i, _):
            base = base_w + i * C
            pltpu.sync_copy(s_hbm.at[pl.ds(base, C)], sidx_v)
            pltpu.sync_copy(t_hbm.at[pl.ds(base, C)], tidx_v)
            pltpu.async_copy(h_hbm.at[sidx_v], rows_v, sem).wait()
            if weighted:
                pltpu.sync_copy(w_hbm.at[pl.ds(base, C)], w_v)

                for b in range(C // 16):
                    w16 = w_v[pl.ds(b * 16, 16)]
                    for l in range(16):
                        r = b * 16 + l
                        ws = w16[l]
                        for j in range(nD8):
                            sl = pl.ds(j * 16, 16)
                            rows_v[r, sl] = rows_v[r, sl] * ws
            pltpu.sync_copy(rows_v, accum_sh.at[tidx_v], add=True)
            return _

        lax.fori_loop(0, nchunks, chunk, 0)
        plsc.subcore_barrier()

        def wb(i, _):
            b = sid + i * _NS

            @pl.when(b < nblk)
            def _do():
                sl = pl.ds(b * 16, 16)
                pltpu.sync_copy(accum_sh.at[sl], zbuf_v)
                pltpu.sync_copy(zbuf_v, out_hbm.at[cid].at[sl])

            return _

        lax.fori_loop(0, blk_iters, wb, 0)

    f = pl.kernel(
        body,
        out_type=jax.ShapeDtypeStruct((_NC, n, D), jnp.float32),
        mesh=_mesh(),
        scratch_types=[
            pltpu.VMEM((C,), jnp.int32),
            pltpu.VMEM((C,), jnp.int32),
            pltpu.VMEM((C,), jnp.float32),
            pltpu.VMEM((C, D), jnp.float32),
            pltpu.VMEM((16, D), jnp.float32),
            pltpu.VMEM_SHARED((n, D), jnp.float32),
            pltpu.SemaphoreType.DMA,
        ],
    )
    return f(h, Sidx, Tidx, w)


@jax.jit
def _sc_count(Tidx, n):
    E = Tidx.shape[0]
    per_w = E // _NW
    C = 128
    assert E % _NW == 0 and per_w % C == 0 and n % (_NS * 8) == 0
    nchunks = per_w // C
    rows_per_sub = n // _NS

    def body(t_hbm, out_hbm, tidx_v, ones_v, zbuf_v, accum_sh):
        cid = lax.axis_index("c")
        sid = lax.axis_index("s")
        wid = cid * _NS + sid
        base_w = wid * per_w

        zero16 = jnp.zeros((16,), jnp.float32)
        one16 = jnp.ones((16,), jnp.float32)
        for j in range(C // 16):
            ones_v[pl.ds(j * 16, 16)] = one16

        def zrow(r, _):
            zbuf_v[pl.ds(r * 16, 16)] = zero16
            return _

        lax.fori_loop(0, rows_per_sub // 16, zrow, 0)
        pltpu.sync_copy(zbuf_v, accum_sh.at[pl.ds(sid * rows_per_sub,
                                                  rows_per_sub)])
        plsc.subcore_barrier()

        def chunk(i, _):
            base = base_w + i * C
            pltpu.sync_copy(t_hbm.at[pl.ds(base, C)], tidx_v)
            pltpu.sync_copy(ones_v, accum_sh.at[tidx_v], add=True)
            return _

        lax.fori_loop(0, nchunks, chunk, 0)
        plsc.subcore_barrier()

        sl = pl.ds(sid * rows_per_sub, rows_per_sub)
        pltpu.sync_copy(accum_sh.at[sl], zbuf_v)
        pltpu.sync_copy(zbuf_v, out_hbm.at[cid].at[sl])

    f = pl.kernel(
        body,
        out_type=jax.ShapeDtypeStruct((_NC, n), jnp.float32),
        mesh=_mesh(),
        scratch_types=[
            pltpu.VMEM((C,), jnp.int32),
            pltpu.VMEM((C,), jnp.float32),
            pltpu.VMEM((n // _NS,), jnp.float32),
            pltpu.VMEM_SHARED((n,), jnp.float32),
        ],
    )
    return f(Tidx)


def _seg_sum_rows(h, Sidx, Tidx, w, n):
    weighted = w is not None
    if not weighted:
        w = jnp.zeros((1,), h.dtype)
        w = jnp.broadcast_to(w, (Sidx.shape[0],))
    p = _sc_rows_agg(h, Sidx, Tidx, w, weighted=weighted)
    return p[0] + p[1]


def _seg_sum_scalar(vals, Tidx, n):
    return jnp.zeros((n,), vals.dtype).at[Tidx].add(vals)


def kernel(x, edge_index, batch, W1, b1, W2, b2, W3, b3, att1, att2,
           Wl1, bl1, Wl2, bl2, Wl3, bl3):
    n = x.shape[0]
    e = edge_index.shape[1]
    k1 = n // 2
    k2 = k1 // 2
    src = edge_index[0]
    dst = edge_index[1]
    f32 = x.dtype
    ones_e = jnp.ones((e,), f32)

    deg0 = _seg_sum_scalar(ones_e, dst, n) + 1.0
    dinv = jax.lax.rsqrt(jnp.maximum(deg0, 1.0))
    g = dinv[:, None] * (x @ W1 + b1)
    aggA = _seg_sum_rows(g, src, dst, None, n)
    h0 = jax.nn.relu(dinv[:, None] * (aggA + g))

    def pool(h, S, T, ew, k, sel_prev, unit_w=False):
        deg = _seg_sum_scalar(ew, T, n)
        agg = _seg_sum_rows(h, S, T, None if unit_w else ew, n) \
            / jnp.maximum(deg, 1e-9)[:, None]
        score = jnp.abs(h - agg).sum(-1)
        msc = score if sel_prev is None else jnp.where(sel_prev, score, -1.0)
        tau = jnp.sort(msc)[n - k]
        sel = msc >= tau
        n0 = jnp.argmax(msc)
        xn = jnp.where(sel[:, None], h * jnp.tanh(score)[:, None], 0.0)
        return sel, n0, xn

    def attention(xn, S, T, ew, sel, n0, att):
        a = xn @ att[:xn.shape[1]]
        b = xn @ att[xn.shape[1]:]
        valid = sel[S] & sel[T]
        logits = jax.nn.leaky_relu(a[S] + b[T], 0.2) + ew
        gm = jnp.max(jnp.where(valid, logits, -jnp.inf))
        ex = jnp.where(valid, jnp.exp(logits - gm), 0.0)
        den = _seg_sum_scalar(ex, T, n)
        new_ew = jnp.where(valid, ex / jnp.maximum(den[T], 1e-16), 0.0)
        return jnp.where(valid, S, n0), jnp.where(valid, T, n0), new_ew

    def readout(xn, k):
        return jnp.concatenate([jnp.max(xn, axis=0), jnp.sum(xn, axis=0) / k])[None, :]

    def gcn_w(xin, S, T, ew, W, b):
        h = xin @ W + b
        deg = _seg_sum_scalar(ew, T, n) + 1.0
        agg = _seg_sum_rows(h, S, T, ew, n) + h
        return jax.nn.relu(agg / deg[:, None])

    sel1, n01, xn1 = pool(h0, src, dst, ones_e, k1, None, unit_w=True)
    S1, T1, ew1 = attention(xn1, src, dst, ones_e, sel1, n01, att1)
    x1 = readout(xn1, k1)

    h1 = gcn_w(xn1, S1, T1, ew1, W2, b2)

    sel2, n02, xn2 = pool(h1, S1, T1, ew1, k2, sel1)
    S2, T2, ew2 = attention(xn2, S1, T1, ew1, sel2, n02, att2)
    x2 = readout(xn2, k2)

    h2 = jnp.where(sel2[:, None], gcn_w(xn2, S2, T2, ew2, W3, b3), 0.0)
    x3 = readout(h2, k2)

    xr = jax.nn.relu(x1) + jax.nn.relu(x2) + jax.nn.relu(x3)
    xr = jax.nn.relu(xr @ Wl1 + bl1)
    xr = jax.nn.relu(xr @ Wl2 + bl2)
    return jax.nn.log_softmax(xr @ Wl3 + bl3, axis=-1)


# speedup vs baseline: 2.0478x; 2.0476x over previous
import functools
import jax
import jax.numpy as jnp
from jax import lax
from jax.experimental import pallas as pl
from jax.experimental.pallas import tpu as pltpu
from jax.experimental.pallas import tpu_sc as plsc

_NC = 2
_NS = 16
_NW = _NC * _NS


def _mesh():
    return plsc.VectorSubcoreMesh(core_axis_name="c", subcore_axis_name="s")


@functools.partial(jax.jit, static_argnames=("weighted",))
def _sc_rows_agg(h, Sidx, Tidx, w, *, weighted):
    n, D = h.shape
    E = Sidx.shape[0]
    per_w = E // _NW
    C = 80
    assert E % _NW == 0 and per_w % C == 0 and n % _NS == 0
    nchunks = per_w // C
    assert n % 16 == 0
    nblk = n // 16
    blk_iters = (nblk + _NS - 1) // _NS
    nD8 = D // 16

    def body(h_hbm, s_hbm, t_hbm, w_hbm, out_hbm, sidx_v, tidx_v, w_v,
             rows_v, zbuf_v, accum_sh, sem):
        cid = lax.axis_index("c")
        sid = lax.axis_index("s")
        wid = cid * _NS + sid
        base_w = wid * per_w

        zero16 = jnp.zeros((16,), jnp.float32)
        for r in range(16):
            for j in range(nD8):
                zbuf_v[r, pl.ds(j * 16, 16)] = zero16

        def zcopy(i, _):
            b = sid + i * _NS

            @pl.when(b < nblk)
            def _do():
                pltpu.sync_copy(zbuf_v, accum_sh.at[pl.ds(b * 16, 16)])

            return _

        lax.fori_loop(0, blk_iters, zcopy, 0)
        plsc.subcore_barrier()

        def chunk(i, _):
            base = base_w + i * C
            pltpu.sync_copy(s_hbm.at[pl.ds(base, C)], sidx_v)
            pltpu.sync_copy(t_hbm.at[pl.ds(base, C)], tidx_v)
            pltpu.async_copy(h_hbm.at[sidx_v], rows_v, sem).wait()
            if weighted:
                pltpu.sync_copy(w_hbm.at[pl.ds(base, C)], w_v)

                for b in range(C // 16):
                    w16 = w_v[pl.ds(b * 16, 16)]
                    for l in range(16):
                        r = b * 16 + l
                        ws = w16[l]
                        for j in range(nD8):
                            sl = pl.ds(j * 16, 16)
                            rows_v[r, sl] = rows_v[r, sl] * ws
            pltpu.sync_copy(rows_v, accum_sh.at[tidx_v], add=True)
            return _

        lax.fori_loop(0, nchunks, chunk, 0)
        plsc.subcore_barrier()

        def wb(i, _):
            b = sid + i * _NS

            @pl.when(b < nblk)
            def _do():
                sl = pl.ds(b * 16, 16)
                pltpu.sync_copy(accum_sh.at[sl], zbuf_v)
                pltpu.sync_copy(zbuf_v, out_hbm.at[cid].at[sl])

            return _

        lax.fori_loop(0, blk_iters, wb, 0)

    f = pl.kernel(
        body,
        out_type=jax.ShapeDtypeStruct((_NC, n, D), jnp.float32),
        mesh=_mesh(),
        scratch_types=[
            pltpu.VMEM((C,), jnp.int32),
            pltpu.VMEM((C,), jnp.int32),
            pltpu.VMEM((C,), jnp.float32),
            pltpu.VMEM((C, D), jnp.float32),
            pltpu.VMEM((16, D), jnp.float32),
            pltpu.VMEM_SHARED((n, D), jnp.float32),
            pltpu.SemaphoreType.DMA,
        ],
    )
    return f(h, Sidx, Tidx, w)


@jax.jit
def _sc_count(Tidx, n):
    E = Tidx.shape[0]
    per_w = E // _NW
    C = 128
    assert E % _NW == 0 and per_w % C == 0 and n % (_NS * 8) == 0
    nchunks = per_w // C
    rows_per_sub = n // _NS

    def body(t_hbm, out_hbm, tidx_v, ones_v, zbuf_v, accum_sh):
        cid = lax.axis_index("c")
        sid = lax.axis_index("s")
        wid = cid * _NS + sid
        base_w = wid * per_w

        zero16 = jnp.zeros((16,), jnp.float32)
        one16 = jnp.ones((16,), jnp.float32)
        for j in range(C // 16):
            ones_v[pl.ds(j * 16, 16)] = one16

        def zrow(r, _):
            zbuf_v[pl.ds(r * 16, 16)] = zero16
            return _

        lax.fori_loop(0, rows_per_sub // 16, zrow, 0)
        pltpu.sync_copy(zbuf_v, accum_sh.at[pl.ds(sid * rows_per_sub,
                                                  rows_per_sub)])
        plsc.subcore_barrier()

        def chunk(i, _):
            base = base_w + i * C
            pltpu.sync_copy(t_hbm.at[pl.ds(base, C)], tidx_v)
            pltpu.sync_copy(ones_v, accum_sh.at[tidx_v], add=True)
            return _

        lax.fori_loop(0, nchunks, chunk, 0)
        plsc.subcore_barrier()

        sl = pl.ds(sid * rows_per_sub, rows_per_sub)
        pltpu.sync_copy(accum_sh.at[sl], zbuf_v)
        pltpu.sync_copy(zbuf_v, out_hbm.at[cid].at[sl])

    f = pl.kernel(
        body,
        out_type=jax.ShapeDtypeStruct((_NC, n), jnp.float32),
        mesh=_mesh(),
        scratch_types=[
            pltpu.VMEM((C,), jnp.int32),
            pltpu.VMEM((C,), jnp.float32),
            pltpu.VMEM((n // _NS,), jnp.float32),
            pltpu.VMEM_SHARED((n,), jnp.float32),
        ],
    )
    return f(Tidx)


def _seg_sum_rows(h, Sidx, Tidx, w, n):
    weighted = w is not None
    if not weighted:
        w = jnp.zeros((1,), h.dtype)
        w = jnp.broadcast_to(w, (Sidx.shape[0],))
    p = _sc_rows_agg(h, Sidx, Tidx, w, weighted=weighted)
    return p[0] + p[1]


def _seg_sum_scalar(vals, Tidx, n):
    return jnp.zeros((n,), vals.dtype).at[Tidx].add(vals)


def kernel(x, edge_index, batch, W1, b1, W2, b2, W3, b3, att1, att2,
           Wl1, bl1, Wl2, bl2, Wl3, bl3):
    n = x.shape[0]
    e = edge_index.shape[1]
    k1 = n // 2
    k2 = k1 // 2
    src = edge_index[0]
    dst = edge_index[1]
    f32 = x.dtype
    ones_e = jnp.ones((e,), f32)

    deg0 = _seg_sum_scalar(ones_e, dst, n) + 1.0
    dinv = jax.lax.rsqrt(jnp.maximum(deg0, 1.0))
    g = dinv[:, None] * (x @ W1 + b1)
    aggA = _seg_sum_rows(g, src, dst, None, n)
    h0 = jax.nn.relu(dinv[:, None] * (aggA + g))

    def pool(h, S, T, ew, k, sel_prev, unit_w=False):
        deg = _seg_sum_scalar(ew, T, n)
        agg = _seg_sum_rows(h, S, T, None if unit_w else ew, n) \
            / jnp.maximum(deg, 1e-9)[:, None]
        score = jnp.abs(h - agg).sum(-1)
        msc = score if sel_prev is None else jnp.where(sel_prev, score, -1.0)
        tau = jnp.sort(msc)[n - k]
        sel = msc >= tau
        n0 = jnp.argmax(msc)
        xn = jnp.where(sel[:, None], h * jnp.tanh(score)[:, None], 0.0)
        return sel, n0, xn

    def attention(xn, S, T, ew, sel, n0, att):
        a = xn @ att[:xn.shape[1]]
        b = xn @ att[xn.shape[1]:]
        valid = sel[S] & sel[T]
        logits = jax.nn.leaky_relu(a[S] + b[T], 0.2) + ew
        gm = jnp.max(jnp.where(valid, logits, -jnp.inf))
        ex = jnp.where(valid, jnp.exp(logits - gm), 0.0)
        den = _seg_sum_scalar(ex, T, n)
        new_ew = jnp.where(valid, ex / jnp.maximum(den[T], 1e-16), 0.0)
        return jnp.where(valid, S, n0), jnp.where(valid, T, n0), new_ew, valid

    def readout(xn, k):
        return jnp.concatenate([jnp.max(xn, axis=0), jnp.sum(xn, axis=0) / k])[None, :]

    def gcn_w(xin, S, T, ew, W, b):
        h = xin @ W + b
        deg = _seg_sum_scalar(ew, T, n) + 1.0
        agg = _seg_sum_rows(h, S, T, ew, n) + h
        return jax.nn.relu(agg / deg[:, None])

    sel1, n01, xn1 = pool(h0, src, dst, ones_e, k1, None, unit_w=True)
    S1, T1, ew1, valid1 = attention(xn1, src, dst, ones_e, sel1, n01, att1)
    x1 = readout(xn1, k1)

    h1 = gcn_w(xn1, src, dst, ew1, W2, b2)

    sel2, n02, xn2 = pool(h1, src, dst, ew1, k2, sel1)
    S2, T2, ew2, _ = attention(xn2, S1, T1, ew1, sel2, n02, att2)
    x2 = readout(xn2, k2)

    h3 = xn2 @ W3 + b3
    degw3 = _seg_sum_scalar(ew2, T2, n) + 1.0
    w3m = jnp.where(valid1, ew2, 0.0)
    csum = jnp.sum(jnp.where(valid1, 0.0, ew2))
    agg3 = _seg_sum_rows(h3, src, dst, w3m, n)
    agg3 = agg3.at[n01].add(csum * h3[n01]) + h3
    h2 = jnp.where(sel2[:, None], jax.nn.relu(agg3 / degw3[:, None]), 0.0)
    x3 = readout(h2, k2)

    xr = jax.nn.relu(x1) + jax.nn.relu(x2) + jax.nn.relu(x3)
    xr = jax.nn.relu(xr @ Wl1 + bl1)
    xr = jax.nn.relu(xr @ Wl2 + bl2)
    return jax.nn.log_softmax(xr @ Wl3 + bl3, axis=-1)


# speedup vs baseline: 14.3690x; 7.0168x over previous
import functools
import jax
import jax.numpy as jnp
from jax import lax
from jax.experimental import pallas as pl
from jax.experimental.pallas import tpu as pltpu
from jax.experimental.pallas import tpu_sc as plsc

_NC = 2
_NS = 16
_NW = _NC * _NS


def _mesh():
    return plsc.VectorSubcoreMesh(core_axis_name="c", subcore_axis_name="s")


@functools.partial(jax.jit, static_argnames=("weighted",))
def _sc_rows_agg(h, Sidx, Tidx, w, *, weighted):
    n, D = h.shape
    E = Sidx.shape[0]
    per_w = E // _NW
    C = 80
    assert E % _NW == 0 and per_w % C == 0 and n % _NS == 0
    nchunks = per_w // C
    assert n % 16 == 0
    nblk = n // 16
    blk_iters = (nblk + _NS - 1) // _NS
    nD8 = D // 16

    def body(h_hbm, s_hbm, t_hbm, w_hbm, out_hbm, sidx_v, tidx_v, w_v,
             rows_v, zbuf_v, accum_sh, sem):
        cid = lax.axis_index("c")
        sid = lax.axis_index("s")
        wid = cid * _NS + sid
        base_w = wid * per_w

        zero16 = jnp.zeros((16,), jnp.float32)
        for r in range(16):
            for j in range(nD8):
                zbuf_v[r, pl.ds(j * 16, 16)] = zero16

        def zcopy(i, _):
            b = sid + i * _NS

            @pl.when(b < nblk)
            def _do():
                pltpu.sync_copy(zbuf_v, accum_sh.at[pl.ds(b * 16, 16)])

            return _

        lax.fori_loop(0, blk_iters, zcopy, 0)
        plsc.subcore_barrier()

        def chunk(i, _):
            base = base_w + i * C
            pltpu.sync_copy(s_hbm.at[pl.ds(base, C)], sidx_v)
            pltpu.sync_copy(t_hbm.at[pl.ds(base, C)], tidx_v)
            pltpu.async_copy(h_hbm.at[sidx_v], rows_v, sem).wait()
            if weighted:
                pltpu.sync_copy(w_hbm.at[pl.ds(base, C)], w_v)

                for b in range(C // 16):
                    w16 = w_v[pl.ds(b * 16, 16)]
                    for l in range(16):
                        r = b * 16 + l
                        ws = w16[l]
                        for j in range(nD8):
                            sl = pl.ds(j * 16, 16)
                            rows_v[r, sl] = rows_v[r, sl] * ws
            pltpu.sync_copy(rows_v, accum_sh.at[tidx_v], add=True)
            return _

        lax.fori_loop(0, nchunks, chunk, 0)
        plsc.subcore_barrier()

        def wb(i, _):
            b = sid + i * _NS

            @pl.when(b < nblk)
            def _do():
                sl = pl.ds(b * 16, 16)
                pltpu.sync_copy(accum_sh.at[sl], zbuf_v)
                pltpu.sync_copy(zbuf_v, out_hbm.at[cid].at[sl])

            return _

        lax.fori_loop(0, blk_iters, wb, 0)

    f = pl.kernel(
        body,
        out_type=jax.ShapeDtypeStruct((_NC, n, D), jnp.float32),
        mesh=_mesh(),
        scratch_types=[
            pltpu.VMEM((C,), jnp.int32),
            pltpu.VMEM((C,), jnp.int32),
            pltpu.VMEM((C,), jnp.float32),
            pltpu.VMEM((C, D), jnp.float32),
            pltpu.VMEM((16, D), jnp.float32),
            pltpu.VMEM_SHARED((n, D), jnp.float32),
            pltpu.SemaphoreType.DMA,
        ],
    )
    return f(h, Sidx, Tidx, w)


def _scalar_accum_helpers(n):
    base_len = (n // (16 * 128)) * 128
    last_len = n - 15 * base_len
    return base_len, last_len


@functools.partial(jax.jit, static_argnames=("stage2",))
def _sc_att(aP, bP, Sg, Tg, Tsc, rprev, exprev, v1, *, stage2):
    n = aP.shape[0]
    E = Sg.shape[0]
    per_w = E // _NW
    C = 80
    assert E % _NW == 0 and per_w % C == 0
    nchunks = per_w // C
    base_len, last_len = _scalar_accum_helpers(n)

    def body(a_hbm, b_hbm, sg_hbm, tg_hbm, tsc_hbm, r_hbm, exp_hbm, v1_hbm,
             ex_hbm, den_hbm, sgi_v, tgi_v, tsi_v, as_v, bs_v, rt_v, exi_v,
             v1_v, ex_v, sct_v, zbuf_v, accum_sh, sem):
        cid = lax.axis_index("c")
        sid = lax.axis_index("s")
        wid = cid * _NS + sid
        base_w = wid * per_w

        zero16 = jnp.zeros((16,), jnp.float32)
        for j in range(max(base_len, last_len) // 16):
            zbuf_v[pl.ds(j * 16, 16)] = zero16

        abase = sid * base_len

        @pl.when(sid < 15)
        def _z0():
            pltpu.sync_copy(zbuf_v.at[pl.ds(0, base_len)],
                            accum_sh.at[pl.ds(abase, base_len)])

        @pl.when(sid == 15)
        def _z1():
            pltpu.sync_copy(zbuf_v.at[pl.ds(0, last_len)],
                            accum_sh.at[pl.ds(15 * base_len, last_len)])

        plsc.subcore_barrier()

        def chunk(i, _):
            base = base_w + i * C
            pltpu.sync_copy(sg_hbm.at[pl.ds(base, C)], sgi_v)
            pltpu.sync_copy(tg_hbm.at[pl.ds(base, C)], tgi_v)
            pltpu.sync_copy(tsc_hbm.at[pl.ds(base, C)], tsi_v)
            pltpu.async_copy(a_hbm.at[sgi_v], as_v, sem).wait()
            pltpu.async_copy(b_hbm.at[tgi_v], bs_v, sem).wait()
            if stage2:
                pltpu.async_copy(r_hbm.at[tgi_v], rt_v, sem).wait()
                pltpu.sync_copy(exp_hbm.at[pl.ds(base, C)], exi_v)
                pltpu.sync_copy(v1_hbm.at[pl.ds(base, C)], v1_v)
            for k in range(C // 16):
                sl = pl.ds(k * 16, 16)
                s = as_v[sl] + bs_v[sl]
                logit = jnp.maximum(s, 0.2 * s)
                if stage2:
                    logit = logit + exi_v[sl] * rt_v[sl]
                else:
                    logit = logit + 1.0
                e = jnp.exp(logit)
                ex_v[sl] = e
                sct_v[sl] = e * v1_v[sl] if stage2 else e
            pltpu.sync_copy(ex_v, ex_hbm.at[pl.ds(base, C)])
            pltpu.sync_copy(sct_v, accum_sh.at[tsi_v], add=True)
            return _

        lax.fori_loop(0, nchunks, chunk, 0)
        plsc.subcore_barrier()

        @pl.when(sid < 15)
        def _w0():
            pltpu.sync_copy(accum_sh.at[pl.ds(abase, base_len)],
                            zbuf_v.at[pl.ds(0, base_len)])
            pltpu.sync_copy(zbuf_v.at[pl.ds(0, base_len)],
                            den_hbm.at[cid].at[pl.ds(abase, base_len)])

        @pl.when(sid == 15)
        def _w1():
            pltpu.sync_copy(accum_sh.at[pl.ds(15 * base_len, last_len)],
                            zbuf_v.at[pl.ds(0, last_len)])
            pltpu.sync_copy(zbuf_v.at[pl.ds(0, last_len)],
                            den_hbm.at[cid].at[pl.ds(15 * base_len, last_len)])

    zlen = 16 * ((max(*_scalar_accum_helpers(n)) + 15) // 16)
    f = pl.kernel(
        body,
        out_type=(jax.ShapeDtypeStruct((E,), jnp.float32),
                  jax.ShapeDtypeStruct((_NC, n), jnp.float32)),
        mesh=_mesh(),
        scratch_types=[
            pltpu.VMEM((C,), jnp.int32),
            pltpu.VMEM((C,), jnp.int32),
            pltpu.VMEM((C,), jnp.int32),
            pltpu.VMEM((C,), jnp.float32),
            pltpu.VMEM((C,), jnp.float32),
            pltpu.VMEM((C,), jnp.float32),
            pltpu.VMEM((C,), jnp.float32),
            pltpu.VMEM((C,), jnp.float32),
            pltpu.VMEM((C,), jnp.float32),
            pltpu.VMEM((C,), jnp.float32),
            pltpu.VMEM((zlen,), jnp.float32),
            pltpu.VMEM_SHARED((n,), jnp.float32),
            pltpu.SemaphoreType.DMA,
        ],
    )
    return f(aP, bP, Sg, Tg, Tsc, rprev, exprev, v1)


@functools.partial(jax.jit, static_argnames=("n",))
def _sc_count(Tidx, n):
    E = Tidx.shape[0]
    per_w = E // _NW
    C = 80
    assert E % _NW == 0 and per_w % C == 0
    nchunks = per_w // C
    base_len, last_len = _scalar_accum_helpers(n)

    def body(t_hbm, out_hbm, tidx_v, ones_v, zbuf_v, accum_sh):
        cid = lax.axis_index("c")
        sid = lax.axis_index("s")
        wid = cid * _NS + sid
        base_w = wid * per_w

        zero16 = jnp.zeros((16,), jnp.float32)
        one16 = jnp.ones((16,), jnp.float32)
        for j in range(C // 16):
            ones_v[pl.ds(j * 16, 16)] = one16
        for j in range(max(base_len, last_len) // 16):
            zbuf_v[pl.ds(j * 16, 16)] = zero16

        abase = sid * base_len

        @pl.when(sid < 15)
        def _z0():
            pltpu.sync_copy(zbuf_v.at[pl.ds(0, base_len)],
                            accum_sh.at[pl.ds(abase, base_len)])

        @pl.when(sid == 15)
        def _z1():
            pltpu.sync_copy(zbuf_v.at[pl.ds(0, last_len)],
                            accum_sh.at[pl.ds(15 * base_len, last_len)])

        plsc.subcore_barrier()

        def chunk(i, _):
            base = base_w + i * C
            pltpu.sync_copy(t_hbm.at[pl.ds(base, C)], tidx_v)
            pltpu.sync_copy(ones_v, accum_sh.at[tidx_v], add=True)
            return _

        lax.fori_loop(0, nchunks, chunk, 0)
        plsc.subcore_barrier()

        @pl.when(sid < 15)
        def _w0():
            pltpu.sync_copy(accum_sh.at[pl.ds(abase, base_len)],
                            zbuf_v.at[pl.ds(0, base_len)])
            pltpu.sync_copy(zbuf_v.at[pl.ds(0, base_len)],
                            out_hbm.at[cid].at[pl.ds(abase, base_len)])

        @pl.when(sid == 15)
        def _w1():
            pltpu.sync_copy(accum_sh.at[pl.ds(15 * base_len, last_len)],
                            zbuf_v.at[pl.ds(0, last_len)])
            pltpu.sync_copy(zbuf_v.at[pl.ds(0, last_len)],
                            out_hbm.at[cid].at[pl.ds(15 * base_len, last_len)])

    zlen = 16 * ((max(base_len, last_len) + 15) // 16)
    f = pl.kernel(
        body,
        out_type=jax.ShapeDtypeStruct((_NC, n), jnp.float32),
        mesh=_mesh(),
        scratch_types=[
            pltpu.VMEM((C,), jnp.int32),
            pltpu.VMEM((C,), jnp.float32),
            pltpu.VMEM((zlen,), jnp.float32),
            pltpu.VMEM_SHARED((n,), jnp.float32),
        ],
    )
    return f(Tidx)


def _seg_sum_rows(h, Sidx, Tidx, w, n):
    weighted = w is not None
    if not weighted:
        w = jnp.zeros((1,), h.dtype)
        w = jnp.broadcast_to(w, (Sidx.shape[0],))
    p = _sc_rows_agg(h, Sidx, Tidx, w, weighted=weighted)
    return p[0] + p[1]


def kernel(x, edge_index, batch, W1, b1, W2, b2, W3, b3, att1, att2,
           Wl1, bl1, Wl2, bl2, Wl3, bl3):
    n = x.shape[0]
    k1 = n // 2
    k2 = k1 // 2
    src = edge_index[0]
    dst = edge_index[1]
    f32 = x.dtype

    def readout(xn, k):
        return jnp.concatenate([jnp.max(xn, axis=0), jnp.sum(xn, axis=0) / k])[None, :]

    def topk_sel(score, k, sel_prev):
        msc = score if sel_prev is None else jnp.where(sel_prev, score, -1.0)
        tau = jnp.sort(msc)[n - k]
        return msc >= tau, jnp.argmax(msc)

    cnt_p = _sc_count(dst, n)
    cnt = cnt_p[0] + cnt_p[1]
    dinv = jax.lax.rsqrt(jnp.maximum(cnt + 1.0, 1.0))
    g = dinv[:, None] * (x @ W1 + b1)
    aggA = _seg_sum_rows(g, src, dst, None, n)
    h0 = jax.nn.relu(dinv[:, None] * (aggA + g))

    agg1 = _seg_sum_rows(h0, src, dst, None, n) \
        / jnp.maximum(cnt, 1e-9)[:, None]
    score1 = jnp.abs(h0 - agg1).sum(-1)
    sel1, n01 = topk_sel(score1, k1, None)
    xn1 = jnp.where(sel1[:, None], h0 * jnp.tanh(score1)[:, None], 0.0)
    x1 = readout(xn1, k1)

    NEG = jnp.float32(-1e9)
    nh = xn1.shape[1]
    aP1 = jnp.where(sel1, xn1 @ att1[:nh], NEG)
    bP1 = jnp.where(sel1, xn1 @ att1[nh:], NEG)
    zeros_e = jnp.zeros((src.shape[0],), f32)
    ex1, den1_p = _sc_att(aP1, bP1, src, dst, dst, aP1, zeros_e, zeros_e,
                          stage2=False)
    den1 = den1_p[0] + den1_p[1]
    r1 = 1.0 / jnp.maximum(den1, 1e-16)
    valid1 = ex1 > 0.0
    S1 = jnp.where(valid1, src, n01)
    T1 = jnp.where(valid1, dst, n01)

    h2in = xn1 @ W2 + b2
    degw2 = den1 * r1 + 1.0
    aggw2 = r1[:, None] * _seg_sum_rows(h2in, src, dst, ex1, n) + h2in
    h1 = jax.nn.relu(aggw2 / degw2[:, None])

    degp2 = den1 * r1
    aggp2 = r1[:, None] * _seg_sum_rows(h1, src, dst, ex1, n) \
        / jnp.maximum(degp2, 1e-9)[:, None]
    score2 = jnp.abs(h1 - aggp2).sum(-1)
    sel2, _ = topk_sel(score2, k2, sel1)
    xn2 = jnp.where(sel2[:, None], h1 * jnp.tanh(score2)[:, None], 0.0)
    x2 = readout(xn2, k2)

    aP2 = jnp.where(sel2, xn2 @ att2[:nh], NEG)
    bP2 = jnp.where(sel2, xn2 @ att2[nh:], NEG)
    v1f = valid1.astype(f32)
    ex2, den2_p = _sc_att(aP2, bP2, S1, T1, dst, r1, ex1, v1f, stage2=True)
    csum = jnp.sum(jnp.where(valid1, 0.0, ex2))
    den2 = (den2_p[0] + den2_p[1]).at[n01].add(csum)
    r2 = 1.0 / jnp.maximum(den2, 1e-16)

    h3 = xn2 @ W3 + b3
    degw3 = den2 * r2 + 1.0
    w3m = jnp.where(valid1, ex2, 0.0)
    agg3 = _seg_sum_rows(h3, src, dst, w3m, n).at[n01].add(csum * h3[n01])
    agg3 = r2[:, None] * agg3 + h3
    h2 = jnp.where(sel2[:, None], jax.nn.relu(agg3 / degw3[:, None]), 0.0)
    x3 = readout(h2, k2)

    xr = jax.nn.relu(x1) + jax.nn.relu(x2) + jax.nn.relu(x3)
    xr = jax.nn.relu(xr @ Wl1 + bl1)
    xr = jax.nn.relu(xr @ Wl2 + bl2)
    return jax.nn.log_softmax(xr @ Wl3 + bl3, axis=-1)


# speedup vs baseline: 14.4946x; 1.0087x over previous
import functools
import jax
import jax.numpy as jnp
from jax import lax
from jax.experimental import pallas as pl
from jax.experimental.pallas import tpu as pltpu
from jax.experimental.pallas import tpu_sc as plsc

_NC = 2
_NS = 16
_NW = _NC * _NS


def _mesh():
    return plsc.VectorSubcoreMesh(core_axis_name="c", subcore_axis_name="s")


@functools.partial(jax.jit, static_argnames=("weighted",))
def _sc_rows_agg(h, Sidx, Tidx, w, *, weighted):
    n, D = h.shape
    E = Sidx.shape[0]
    per_w = E // _NW
    C = 80
    assert E % _NW == 0 and per_w % C == 0 and n % _NS == 0
    nchunks = per_w // C
    assert n % 16 == 0
    nblk = n // 16
    blk_iters = (nblk + _NS - 1) // _NS
    nD8 = D // 16

    def body(h_hbm, s_hbm, t_hbm, w_hbm, out_hbm, sidx_v, tidx_v, w_v,
             rows_v, zbuf_v, accum_sh, sem):
        cid = lax.axis_index("c")
        sid = lax.axis_index("s")
        wid = cid * _NS + sid
        base_w = wid * per_w

        zero16 = jnp.zeros((16,), jnp.float32)
        for r in range(16):
            for j in range(nD8):
                zbuf_v[r, pl.ds(j * 16, 16)] = zero16

        def zcopy(i, _):
            b = sid + i * _NS

            @pl.when(b < nblk)
            def _do():
                pltpu.sync_copy(zbuf_v, accum_sh.at[pl.ds(b * 16, 16)])

            return _

        lax.fori_loop(0, blk_iters, zcopy, 0)
        plsc.subcore_barrier()

        def chunk(i, _):
            base = base_w + i * C
            pltpu.sync_copy(s_hbm.at[pl.ds(base, C)], sidx_v)
            pltpu.sync_copy(t_hbm.at[pl.ds(base, C)], tidx_v)
            pltpu.async_copy(h_hbm.at[sidx_v], rows_v, sem).wait()
            if weighted:
                pltpu.sync_copy(w_hbm.at[pl.ds(base, C)], w_v)

                for b in range(C // 16):
                    w16 = w_v[pl.ds(b * 16, 16)]
                    for l in range(16):
                        r = b * 16 + l
                        ws = w16[l]
                        for j in range(nD8):
                            sl = pl.ds(j * 16, 16)
                            rows_v[r, sl] = rows_v[r, sl] * ws
            pltpu.sync_copy(rows_v, accum_sh.at[tidx_v], add=True)
            return _

        lax.fori_loop(0, nchunks, chunk, 0)
        plsc.subcore_barrier()

        def wb(i, _):
            b = sid + i * _NS

            @pl.when(b < nblk)
            def _do():
                sl = pl.ds(b * 16, 16)
                pltpu.sync_copy(accum_sh.at[sl], zbuf_v)
                pltpu.sync_copy(zbuf_v, out_hbm.at[cid].at[sl])

            return _

        lax.fori_loop(0, blk_iters, wb, 0)

    f = pl.kernel(
        body,
        out_type=jax.ShapeDtypeStruct((_NC, n, D), jnp.float32),
        mesh=_mesh(),
        scratch_types=[
            pltpu.VMEM((C,), jnp.int32),
            pltpu.VMEM((C,), jnp.int32),
            pltpu.VMEM((C,), jnp.float32),
            pltpu.VMEM((C, D), jnp.float32),
            pltpu.VMEM((16, D), jnp.float32),
            pltpu.VMEM_SHARED((n, D), jnp.float32),
            pltpu.SemaphoreType.DMA,
        ],
    )
    return f(h, Sidx, Tidx, w)


def _scalar_accum_helpers(n):
    base_len = (n // (16 * 128)) * 128
    last_len = n - 15 * base_len
    return base_len, last_len


@functools.partial(jax.jit, static_argnames=("stage2",))
def _sc_att(aP, bP, Sg, Tg, Tsc, rprev, exprev, v1, *, stage2):
    n = aP.shape[0]
    E = Sg.shape[0]
    per_w = E // _NW
    C = 80
    assert E % _NW == 0 and per_w % C == 0
    nchunks = per_w // C
    base_len, last_len = _scalar_accum_helpers(n)

    def body(a_hbm, b_hbm, sg_hbm, tg_hbm, tsc_hbm, r_hbm, exp_hbm, v1_hbm,
             ex_hbm, den_hbm, sgi_v, tgi_v, tsi_v, as_v, bs_v, rt_v, exi_v,
             v1_v, ex_v, sct_v, zbuf_v, accum_sh, sem):
        cid = lax.axis_index("c")
        sid = lax.axis_index("s")
        wid = cid * _NS + sid
        base_w = wid * per_w

        zero16 = jnp.zeros((16,), jnp.float32)
        for j in range(max(base_len, last_len) // 16):
            zbuf_v[pl.ds(j * 16, 16)] = zero16

        abase = sid * base_len

        @pl.when(sid < 15)
        def _z0():
            pltpu.sync_copy(zbuf_v.at[pl.ds(0, base_len)],
                            accum_sh.at[pl.ds(abase, base_len)])

        @pl.when(sid == 15)
        def _z1():
            pltpu.sync_copy(zbuf_v.at[pl.ds(0, last_len)],
                            accum_sh.at[pl.ds(15 * base_len, last_len)])

        plsc.subcore_barrier()

        def chunk(i, _):
            base = base_w + i * C
            pltpu.sync_copy(sg_hbm.at[pl.ds(base, C)], sgi_v)
            pltpu.sync_copy(tg_hbm.at[pl.ds(base, C)], tgi_v)
            pltpu.sync_copy(tsc_hbm.at[pl.ds(base, C)], tsi_v)
            pltpu.async_copy(a_hbm.at[sgi_v], as_v, sem).wait()
            pltpu.async_copy(b_hbm.at[tgi_v], bs_v, sem).wait()
            if stage2:
                pltpu.async_copy(r_hbm.at[tgi_v], rt_v, sem).wait()
                pltpu.sync_copy(exp_hbm.at[pl.ds(base, C)], exi_v)
                pltpu.sync_copy(v1_hbm.at[pl.ds(base, C)], v1_v)
            for k in range(C // 16):
                sl = pl.ds(k * 16, 16)
                s = as_v[sl] + bs_v[sl]
                logit = jnp.maximum(s, 0.2 * s)
                if stage2:
                    logit = logit + exi_v[sl] * rt_v[sl]
                else:
                    logit = logit + 1.0
                e = jnp.exp(logit)
                ex_v[sl] = e
                sct_v[sl] = e * v1_v[sl] if stage2 else e
            pltpu.sync_copy(ex_v, ex_hbm.at[pl.ds(base, C)])
            pltpu.sync_copy(sct_v, accum_sh.at[tsi_v], add=True)
            return _

        lax.fori_loop(0, nchunks, chunk, 0)
        plsc.subcore_barrier()

        @pl.when(sid < 15)
        def _w0():
            pltpu.sync_copy(accum_sh.at[pl.ds(abase, base_len)],
                            zbuf_v.at[pl.ds(0, base_len)])
            pltpu.sync_copy(zbuf_v.at[pl.ds(0, base_len)],
                            den_hbm.at[cid].at[pl.ds(abase, base_len)])

        @pl.when(sid == 15)
        def _w1():
            pltpu.sync_copy(accum_sh.at[pl.ds(15 * base_len, last_len)],
                            zbuf_v.at[pl.ds(0, last_len)])
            pltpu.sync_copy(zbuf_v.at[pl.ds(0, last_len)],
                            den_hbm.at[cid].at[pl.ds(15 * base_len, last_len)])

    zlen = 16 * ((max(*_scalar_accum_helpers(n)) + 15) // 16)
    f = pl.kernel(
        body,
        out_type=(jax.ShapeDtypeStruct((E,), jnp.float32),
                  jax.ShapeDtypeStruct((_NC, n), jnp.float32)),
        mesh=_mesh(),
        scratch_types=[
            pltpu.VMEM((C,), jnp.int32),
            pltpu.VMEM((C,), jnp.int32),
            pltpu.VMEM((C,), jnp.int32),
            pltpu.VMEM((C,), jnp.float32),
            pltpu.VMEM((C,), jnp.float32),
            pltpu.VMEM((C,), jnp.float32),
            pltpu.VMEM((C,), jnp.float32),
            pltpu.VMEM((C,), jnp.float32),
            pltpu.VMEM((C,), jnp.float32),
            pltpu.VMEM((C,), jnp.float32),
            pltpu.VMEM((zlen,), jnp.float32),
            pltpu.VMEM_SHARED((n,), jnp.float32),
            pltpu.SemaphoreType.DMA,
        ],
    )
    return f(aP, bP, Sg, Tg, Tsc, rprev, exprev, v1)


@functools.partial(jax.jit, static_argnames=("n",))
def _sc_count(Tidx, n):
    E = Tidx.shape[0]
    per_w = E // _NW
    C = 80
    assert E % _NW == 0 and per_w % C == 0
    nchunks = per_w // C
    base_len, last_len = _scalar_accum_helpers(n)

    def body(t_hbm, out_hbm, tidx_v, ones_v, zbuf_v, accum_sh):
        cid = lax.axis_index("c")
        sid = lax.axis_index("s")
        wid = cid * _NS + sid
        base_w = wid * per_w

        zero16 = jnp.zeros((16,), jnp.float32)
        one16 = jnp.ones((16,), jnp.float32)
        for j in range(C // 16):
            ones_v[pl.ds(j * 16, 16)] = one16
        for j in range(max(base_len, last_len) // 16):
            zbuf_v[pl.ds(j * 16, 16)] = zero16

        abase = sid * base_len

        @pl.when(sid < 15)
        def _z0():
            pltpu.sync_copy(zbuf_v.at[pl.ds(0, base_len)],
                            accum_sh.at[pl.ds(abase, base_len)])

        @pl.when(sid == 15)
        def _z1():
            pltpu.sync_copy(zbuf_v.at[pl.ds(0, last_len)],
                            accum_sh.at[pl.ds(15 * base_len, last_len)])

        plsc.subcore_barrier()

        def chunk(i, _):
            base = base_w + i * C
            pltpu.sync_copy(t_hbm.at[pl.ds(base, C)], tidx_v)
            pltpu.sync_copy(ones_v, accum_sh.at[tidx_v], add=True)
            return _

        lax.fori_loop(0, nchunks, chunk, 0)
        plsc.subcore_barrier()

        @pl.when(sid < 15)
        def _w0():
            pltpu.sync_copy(accum_sh.at[pl.ds(abase, base_len)],
                            zbuf_v.at[pl.ds(0, base_len)])
            pltpu.sync_copy(zbuf_v.at[pl.ds(0, base_len)],
                            out_hbm.at[cid].at[pl.ds(abase, base_len)])

        @pl.when(sid == 15)
        def _w1():
            pltpu.sync_copy(accum_sh.at[pl.ds(15 * base_len, last_len)],
                            zbuf_v.at[pl.ds(0, last_len)])
            pltpu.sync_copy(zbuf_v.at[pl.ds(0, last_len)],
                            out_hbm.at[cid].at[pl.ds(15 * base_len, last_len)])

    zlen = 16 * ((max(base_len, last_len) + 15) // 16)
    f = pl.kernel(
        body,
        out_type=jax.ShapeDtypeStruct((_NC, n), jnp.float32),
        mesh=_mesh(),
        scratch_types=[
            pltpu.VMEM((C,), jnp.int32),
            pltpu.VMEM((C,), jnp.float32),
            pltpu.VMEM((zlen,), jnp.float32),
            pltpu.VMEM_SHARED((n,), jnp.float32),
        ],
    )
    return f(Tidx)


@functools.partial(jax.jit, static_argnames=("k",))
def _tc_topk_threshold(msc_pad, k):
    R, L = msc_pad.shape

    def body(v_ref, tau_ref, idx_ref):
        v = v_ref[...]
        lo = jnp.min(v)
        hi = jnp.max(v)

        def it(_, carry):
            lo, hi = carry
            mid = 0.5 * (lo + hi)
            cnt = jnp.sum((v >= mid).astype(jnp.float32))
            pred = cnt >= k
            return jnp.where(pred, mid, lo), jnp.where(pred, hi, mid)

        lo, hi = lax.fori_loop(0, 45, it, (lo, hi))
        tau_ref[...] = jnp.reshape(lo, (1, 1))
        mx = jnp.max(v)
        row = lax.broadcasted_iota(jnp.int32, (R, L), 0)
        col = lax.broadcasted_iota(jnp.int32, (R, L), 1)
        flat = row * L + col
        idx_ref[...] = jnp.reshape(jnp.min(jnp.where(v == mx, flat, R * L)),
                                   (1, 1))

    return pl.pallas_call(
        body,
        out_shape=(jax.ShapeDtypeStruct((1, 1), jnp.float32),
                   jax.ShapeDtypeStruct((1, 1), jnp.int32)),
    )(msc_pad)


def _mm_bias(X, W, b):
    n, d = X.shape
    m = W.shape[1]
    blk = 1000
    assert n % blk == 0

    def body(x_ref, w_ref, b_ref, o_ref):
        o_ref[...] = jnp.dot(x_ref[...], w_ref[...],
                             preferred_element_type=jnp.float32) + b_ref[...]

    return pl.pallas_call(
        body,
        grid=(n // blk,),
        in_specs=[
            pl.BlockSpec((blk, d), lambda i: (i, 0)),
            pl.BlockSpec((d, m), lambda i: (0, 0)),
            pl.BlockSpec((1, m), lambda i: (0, 0)),
        ],
        out_specs=pl.BlockSpec((blk, m), lambda i: (i, 0)),
        out_shape=jax.ShapeDtypeStruct((n, m), jnp.float32),
    )(X, W, b.reshape(1, m))


def _seg_sum_rows(h, Sidx, Tidx, w, n):
    weighted = w is not None
    if not weighted:
        w = jnp.zeros((1,), h.dtype)
        w = jnp.broadcast_to(w, (Sidx.shape[0],))
    p = _sc_rows_agg(h, Sidx, Tidx, w, weighted=weighted)
    return p[0] + p[1]


def kernel(x, edge_index, batch, W1, b1, W2, b2, W3, b3, att1, att2,
           Wl1, bl1, Wl2, bl2, Wl3, bl3):
    n = x.shape[0]
    k1 = n // 2
    k2 = k1 // 2
    src = edge_index[0]
    dst = edge_index[1]
    f32 = x.dtype

    def readout(xn, k):
        return jnp.concatenate([jnp.max(xn, axis=0), jnp.sum(xn, axis=0) / k])[None, :]

    def topk_sel(score, k, sel_prev):
        msc = score if sel_prev is None else jnp.where(sel_prev, score, -1.0)
        npad = ((n + 127) // 128) * 128
        mp = jnp.pad(msc, (0, npad - n), constant_values=-3.0)
        tau, n0 = _tc_topk_threshold(mp.reshape(npad // 128, 128), k)
        return msc >= tau[0, 0], n0[0, 0]

    cnt_p = _sc_count(dst, n)
    cnt = cnt_p[0] + cnt_p[1]
    dinv = jax.lax.rsqrt(jnp.maximum(cnt + 1.0, 1.0))
    g = dinv[:, None] * _mm_bias(x, W1, b1)
    aggA = _seg_sum_rows(g, src, dst, None, n)
    h0 = jax.nn.relu(dinv[:, None] * (aggA + g))

    agg1 = _seg_sum_rows(h0, src, dst, None, n) \
        / jnp.maximum(cnt, 1e-9)[:, None]
    score1 = jnp.abs(h0 - agg1).sum(-1)
    sel1, n01 = topk_sel(score1, k1, None)
    xn1 = jnp.where(sel1[:, None], h0 * jnp.tanh(score1)[:, None], 0.0)
    x1 = readout(xn1, k1)

    NEG = jnp.float32(-1e9)
    nh = xn1.shape[1]
    ab1 = _mm_bias(xn1, jnp.stack([att1[:nh], att1[nh:]], axis=1),
                   jnp.zeros((2,), f32))
    aP1 = jnp.where(sel1, ab1[:, 0], NEG)
    bP1 = jnp.where(sel1, ab1[:, 1], NEG)
    zeros_e = jnp.zeros((src.shape[0],), f32)
    ex1, den1_p = _sc_att(aP1, bP1, src, dst, dst, aP1, zeros_e, zeros_e,
                          stage2=False)
    den1 = den1_p[0] + den1_p[1]
    r1 = 1.0 / jnp.maximum(den1, 1e-16)
    valid1 = ex1 > 0.0
    S1 = jnp.where(valid1, src, n01)
    T1 = jnp.where(valid1, dst, n01)

    h2in = _mm_bias(xn1, W2, b2)
    degw2 = den1 * r1 + 1.0
    aggw2 = r1[:, None] * _seg_sum_rows(h2in, src, dst, ex1, n) + h2in
    h1 = jax.nn.relu(aggw2 / degw2[:, None])

    degp2 = den1 * r1
    aggp2 = r1[:, None] * _seg_sum_rows(h1, src, dst, ex1, n) \
        / jnp.maximum(degp2, 1e-9)[:, None]
    score2 = jnp.abs(h1 - aggp2).sum(-1)
    sel2, _ = topk_sel(score2, k2, sel1)
    xn2 = jnp.where(sel2[:, None], h1 * jnp.tanh(score2)[:, None], 0.0)
    x2 = readout(xn2, k2)

    ab2 = _mm_bias(xn2, jnp.stack([att2[:nh], att2[nh:]], axis=1),
                   jnp.zeros((2,), f32))
    aP2 = jnp.where(sel2, ab2[:, 0], NEG)
    bP2 = jnp.where(sel2, ab2[:, 1], NEG)
    v1f = valid1.astype(f32)
    ex2, den2_p = _sc_att(aP2, bP2, S1, T1, dst, r1, ex1, v1f, stage2=True)
    csum = jnp.sum(jnp.where(valid1, 0.0, ex2))
    den2 = (den2_p[0] + den2_p[1]).at[n01].add(csum)
    r2 = 1.0 / jnp.maximum(den2, 1e-16)

    h3 = _mm_bias(xn2, W3, b3)
    degw3 = den2 * r2 + 1.0
    w3m = jnp.where(valid1, ex2, 0.0)
    agg3 = _seg_sum_rows(h3, src, dst, w3m, n).at[n01].add(csum * h3[n01])
    agg3 = r2[:, None] * agg3 + h3
    h2 = jnp.where(sel2[:, None], jax.nn.relu(agg3 / degw3[:, None]), 0.0)
    x3 = readout(h2, k2)

    xr = jax.nn.relu(x1) + jax.nn.relu(x2) + jax.nn.relu(x3)
    xr = jax.nn.relu(xr @ Wl1 + bl1)
    xr = jax.nn.relu(xr @ Wl2 + bl2)
    return jax.nn.log_softmax(xr @ Wl3 + bl3, axis=-1)


# speedup vs baseline: 17.4598x; 1.2046x over previous
import functools
import jax
import jax.numpy as jnp
from jax import lax
from jax.experimental import pallas as pl
from jax.experimental.pallas import tpu as pltpu
from jax.experimental.pallas import tpu_sc as plsc

_NC = 2
_NS = 16
_NW = _NC * _NS


def _mesh():
    return plsc.VectorSubcoreMesh(core_axis_name="c", subcore_axis_name="s")


@functools.partial(jax.jit, static_argnames=("weighted",))
def _sc_rows_agg(h, Sidx, Tidx, w, *, weighted):
    n, D = h.shape
    E = Sidx.shape[0]
    per_w = E // _NW
    C = 80
    assert E % _NW == 0 and per_w % C == 0 and n % _NS == 0
    nchunks = per_w // C
    assert n % 16 == 0
    nblk = n // 16
    blk_iters = (nblk + _NS - 1) // _NS
    nD8 = D // 16

    def body(h_hbm, s_hbm, t_hbm, w_hbm, out_hbm, sidx_v, tidx_v, w_v,
             rows_v, zbuf_v, accum_sh, sem):
        cid = lax.axis_index("c")
        sid = lax.axis_index("s")
        wid = cid * _NS + sid
        base_w = wid * per_w

        zero16 = jnp.zeros((16,), jnp.float32)
        for r in range(16):
            for j in range(nD8):
                zbuf_v[r, pl.ds(j * 16, 16)] = zero16

        def zcopy(i, _):
            b = sid + i * _NS

            @pl.when(b < nblk)
            def _do():
                pltpu.sync_copy(zbuf_v, accum_sh.at[pl.ds(b * 16, 16)])

            return _

        lax.fori_loop(0, blk_iters, zcopy, 0)
        plsc.subcore_barrier()

        def chunk(i, _):
            base = base_w + i * C
            pltpu.sync_copy(s_hbm.at[pl.ds(base, C)], sidx_v)
            pltpu.sync_copy(t_hbm.at[pl.ds(base, C)], tidx_v)
            pltpu.async_copy(h_hbm.at[sidx_v], rows_v, sem).wait()
            if weighted:
                pltpu.sync_copy(w_hbm.at[pl.ds(base, C)], w_v)

                for b in range(C // 16):
                    w16 = w_v[pl.ds(b * 16, 16)]
                    for l in range(16):
                        r = b * 16 + l
                        ws = w16[l]
                        for j in range(nD8):
                            sl = pl.ds(j * 16, 16)
                            rows_v[r, sl] = rows_v[r, sl] * ws
            pltpu.sync_copy(rows_v, accum_sh.at[tidx_v], add=True)
            return _

        lax.fori_loop(0, nchunks, chunk, 0)
        plsc.subcore_barrier()

        def wb(i, _):
            b = sid + i * _NS

            @pl.when(b < nblk)
            def _do():
                sl = pl.ds(b * 16, 16)
                pltpu.sync_copy(accum_sh.at[sl], zbuf_v)
                pltpu.sync_copy(zbuf_v, out_hbm.at[cid].at[sl])

            return _

        lax.fori_loop(0, blk_iters, wb, 0)

    f = pl.kernel(
        body,
        out_type=jax.ShapeDtypeStruct((_NC, n, D), jnp.float32),
        mesh=_mesh(),
        scratch_types=[
            pltpu.VMEM((C,), jnp.int32),
            pltpu.VMEM((C,), jnp.int32),
            pltpu.VMEM((C,), jnp.float32),
            pltpu.VMEM((C, D), jnp.float32),
            pltpu.VMEM((16, D), jnp.float32),
            pltpu.VMEM_SHARED((n, D), jnp.float32),
            pltpu.SemaphoreType.DMA,
        ],
    )
    return f(h, Sidx, Tidx, w)


def _scalar_accum_helpers(n):
    base_len = (n // (16 * 128)) * 128
    last_len = n - 15 * base_len
    return base_len, last_len


@functools.partial(jax.jit, static_argnames=("stage2",))
def _sc_att(aP, bP, Sg, Tg, Tsc, rprev, exprev, v1, *, stage2):
    n = aP.shape[0]
    E = Sg.shape[0]
    per_w = E // _NW
    C = 80
    assert E % _NW == 0 and per_w % C == 0
    nchunks = per_w // C
    base_len, last_len = _scalar_accum_helpers(n)

    def body(a_hbm, b_hbm, sg_hbm, tg_hbm, tsc_hbm, r_hbm, exp_hbm, v1_hbm,
             ex_hbm, den_hbm, sgi_v, tgi_v, tsi_v, as_v, bs_v, rt_v, exi_v,
             v1_v, ex_v, sct_v, zbuf_v, accum_sh, sem):
        cid = lax.axis_index("c")
        sid = lax.axis_index("s")
        wid = cid * _NS + sid
        base_w = wid * per_w

        zero16 = jnp.zeros((16,), jnp.float32)
        for j in range(max(base_len, last_len) // 16):
            zbuf_v[pl.ds(j * 16, 16)] = zero16

        abase = sid * base_len

        @pl.when(sid < 15)
        def _z0():
            pltpu.sync_copy(zbuf_v.at[pl.ds(0, base_len)],
                            accum_sh.at[pl.ds(abase, base_len)])

        @pl.when(sid == 15)
        def _z1():
            pltpu.sync_copy(zbuf_v.at[pl.ds(0, last_len)],
                            accum_sh.at[pl.ds(15 * base_len, last_len)])

        plsc.subcore_barrier()

        def chunk(i, _):
            base = base_w + i * C
            pltpu.sync_copy(sg_hbm.at[pl.ds(base, C)], sgi_v)
            pltpu.sync_copy(tg_hbm.at[pl.ds(base, C)], tgi_v)
            pltpu.sync_copy(tsc_hbm.at[pl.ds(base, C)], tsi_v)
            pltpu.async_copy(a_hbm.at[sgi_v], as_v, sem).wait()
            pltpu.async_copy(b_hbm.at[tgi_v], bs_v, sem).wait()
            if stage2:
                pltpu.async_copy(r_hbm.at[tgi_v], rt_v, sem).wait()
                pltpu.sync_copy(exp_hbm.at[pl.ds(base, C)], exi_v)
                pltpu.sync_copy(v1_hbm.at[pl.ds(base, C)], v1_v)
            for k in range(C // 16):
                sl = pl.ds(k * 16, 16)
                s = as_v[sl] + bs_v[sl]
                logit = jnp.maximum(s, 0.2 * s)
                if stage2:
                    logit = logit + exi_v[sl] * rt_v[sl]
                else:
                    logit = logit + 1.0
                e = jnp.exp(logit)
                ex_v[sl] = e
                sct_v[sl] = e * v1_v[sl] if stage2 else e
            pltpu.sync_copy(ex_v, ex_hbm.at[pl.ds(base, C)])
            pltpu.sync_copy(sct_v, accum_sh.at[tsi_v], add=True)
            return _

        lax.fori_loop(0, nchunks, chunk, 0)
        plsc.subcore_barrier()

        @pl.when(sid < 15)
        def _w0():
            pltpu.sync_copy(accum_sh.at[pl.ds(abase, base_len)],
                            zbuf_v.at[pl.ds(0, base_len)])
            pltpu.sync_copy(zbuf_v.at[pl.ds(0, base_len)],
                            den_hbm.at[cid].at[pl.ds(abase, base_len)])

        @pl.when(sid == 15)
        def _w1():
            pltpu.sync_copy(accum_sh.at[pl.ds(15 * base_len, last_len)],
                            zbuf_v.at[pl.ds(0, last_len)])
            pltpu.sync_copy(zbuf_v.at[pl.ds(0, last_len)],
                            den_hbm.at[cid].at[pl.ds(15 * base_len, last_len)])

    zlen = 16 * ((max(*_scalar_accum_helpers(n)) + 15) // 16)
    f = pl.kernel(
        body,
        out_type=(jax.ShapeDtypeStruct((E,), jnp.float32),
                  jax.ShapeDtypeStruct((_NC, n), jnp.float32)),
        mesh=_mesh(),
        scratch_types=[
            pltpu.VMEM((C,), jnp.int32),
            pltpu.VMEM((C,), jnp.int32),
            pltpu.VMEM((C,), jnp.int32),
            pltpu.VMEM((C,), jnp.float32),
            pltpu.VMEM((C,), jnp.float32),
            pltpu.VMEM((C,), jnp.float32),
            pltpu.VMEM((C,), jnp.float32),
            pltpu.VMEM((C,), jnp.float32),
            pltpu.VMEM((C,), jnp.float32),
            pltpu.VMEM((C,), jnp.float32),
            pltpu.VMEM((zlen,), jnp.float32),
            pltpu.VMEM_SHARED((n,), jnp.float32),
            pltpu.SemaphoreType.DMA,
        ],
    )
    return f(aP, bP, Sg, Tg, Tsc, rprev, exprev, v1)


@functools.partial(jax.jit, static_argnames=("n",))
def _sc_count(Tidx, n):
    E = Tidx.shape[0]
    per_w = E // _NW
    C = 80
    assert E % _NW == 0 and per_w % C == 0
    nchunks = per_w // C
    base_len, last_len = _scalar_accum_helpers(n)

    def body(t_hbm, out_hbm, tidx_v, ones_v, zbuf_v, accum_sh):
        cid = lax.axis_index("c")
        sid = lax.axis_index("s")
        wid = cid * _NS + sid
        base_w = wid * per_w

        zero16 = jnp.zeros((16,), jnp.float32)
        one16 = jnp.ones((16,), jnp.float32)
        for j in range(C // 16):
            ones_v[pl.ds(j * 16, 16)] = one16
        for j in range(max(base_len, last_len) // 16):
            zbuf_v[pl.ds(j * 16, 16)] = zero16

        abase = sid * base_len

        @pl.when(sid < 15)
        def _z0():
            pltpu.sync_copy(zbuf_v.at[pl.ds(0, base_len)],
                            accum_sh.at[pl.ds(abase, base_len)])

        @pl.when(sid == 15)
        def _z1():
            pltpu.sync_copy(zbuf_v.at[pl.ds(0, last_len)],
                            accum_sh.at[pl.ds(15 * base_len, last_len)])

        plsc.subcore_barrier()

        def chunk(i, _):
            base = base_w + i * C
            pltpu.sync_copy(t_hbm.at[pl.ds(base, C)], tidx_v)
            pltpu.sync_copy(ones_v, accum_sh.at[tidx_v], add=True)
            return _

        lax.fori_loop(0, nchunks, chunk, 0)
        plsc.subcore_barrier()

        @pl.when(sid < 15)
        def _w0():
            pltpu.sync_copy(accum_sh.at[pl.ds(abase, base_len)],
                            zbuf_v.at[pl.ds(0, base_len)])
            pltpu.sync_copy(zbuf_v.at[pl.ds(0, base_len)],
                            out_hbm.at[cid].at[pl.ds(abase, base_len)])

        @pl.when(sid == 15)
        def _w1():
            pltpu.sync_copy(accum_sh.at[pl.ds(15 * base_len, last_len)],
                            zbuf_v.at[pl.ds(0, last_len)])
            pltpu.sync_copy(zbuf_v.at[pl.ds(0, last_len)],
                            out_hbm.at[cid].at[pl.ds(15 * base_len, last_len)])

    zlen = 16 * ((max(base_len, last_len) + 15) // 16)
    f = pl.kernel(
        body,
        out_type=jax.ShapeDtypeStruct((_NC, n), jnp.float32),
        mesh=_mesh(),
        scratch_types=[
            pltpu.VMEM((C,), jnp.int32),
            pltpu.VMEM((C,), jnp.float32),
            pltpu.VMEM((zlen,), jnp.float32),
            pltpu.VMEM_SHARED((n,), jnp.float32),
        ],
    )
    return f(Tidx)


@functools.partial(jax.jit, static_argnames=("k",))
def _tc_topk_threshold(msc_pad, k):
    R, L = msc_pad.shape

    def body(v_ref, tau_ref, idx_ref):
        v = v_ref[...]
        lo = jnp.min(v)
        hi = jnp.max(v)

        def it(_, carry):
            lo, hi = carry
            mid = 0.5 * (lo + hi)
            cnt = jnp.sum((v >= mid).astype(jnp.float32))
            pred = cnt >= k
            return jnp.where(pred, mid, lo), jnp.where(pred, hi, mid)

        lo, hi = lax.fori_loop(0, 45, it, (lo, hi))
        tau_ref[...] = jnp.reshape(lo, (1, 1))
        mx = jnp.max(v)
        row = lax.broadcasted_iota(jnp.int32, (R, L), 0)
        col = lax.broadcasted_iota(jnp.int32, (R, L), 1)
        flat = row * L + col
        idx_ref[...] = jnp.reshape(jnp.min(jnp.where(v == mx, flat, R * L)),
                                   (1, 1))

    return pl.pallas_call(
        body,
        out_shape=(jax.ShapeDtypeStruct((1, 1), jnp.float32),
                   jax.ShapeDtypeStruct((1, 1), jnp.int32)),
    )(msc_pad)


def _mm_bias(X, W, b):
    n, d = X.shape
    m = W.shape[1]
    blk = 1000
    assert n % blk == 0

    def body(x_ref, w_ref, b_ref, o_ref):
        o_ref[...] = jnp.dot(x_ref[...], w_ref[...],
                             preferred_element_type=jnp.float32) + b_ref[...]

    return pl.pallas_call(
        body,
        grid=(n // blk,),
        in_specs=[
            pl.BlockSpec((blk, d), lambda i: (i, 0)),
            pl.BlockSpec((d, m), lambda i: (0, 0)),
            pl.BlockSpec((1, m), lambda i: (0, 0)),
        ],
        out_specs=pl.BlockSpec((blk, m), lambda i: (i, 0)),
        out_shape=jax.ShapeDtypeStruct((n, m), jnp.float32),
    )(X, W, b.reshape(1, m))


def _seg_sum_rows(h, Sidx, Tidx, w, n):
    weighted = w is not None
    if not weighted:
        w = jnp.zeros((1,), h.dtype)
        w = jnp.broadcast_to(w, (Sidx.shape[0],))
    p = _sc_rows_agg(h, Sidx, Tidx, w, weighted=weighted)
    return p[0] + p[1]


def kernel(x, edge_index, batch, W1, b1, W2, b2, W3, b3, att1, att2,
           Wl1, bl1, Wl2, bl2, Wl3, bl3):
    n = x.shape[0]
    k1 = n // 2
    k2 = k1 // 2
    src = edge_index[0]
    dst = edge_index[1]
    f32 = x.dtype

    def readout(xn, k):
        return jnp.concatenate([jnp.max(xn, axis=0), jnp.sum(xn, axis=0) / k])[None, :]

    def topk_sel(score, k, sel_prev):
        msc = score if sel_prev is None else jnp.where(sel_prev, score, -1.0)
        npad = ((n + 127) // 128) * 128
        mp = jnp.pad(msc, (0, npad - n), constant_values=-3.0)
        tau, n0 = _tc_topk_threshold(mp.reshape(npad // 128, 128), k)
        return msc >= tau[0, 0], n0[0, 0]

    cnt_p = _sc_count(dst, n)
    cnt = cnt_p[0] + cnt_p[1]
    dinv = jax.lax.rsqrt(jnp.maximum(cnt + 1.0, 1.0))
    g = dinv[:, None] * _mm_bias(x, W1, b1)
    aggA = _seg_sum_rows(g, src, dst, None, n)
    h0 = jax.nn.relu(dinv[:, None] * (aggA + g))

    agg1 = _seg_sum_rows(h0, src, dst, None, n) \
        / jnp.maximum(cnt, 1e-9)[:, None]
    score1 = jnp.abs(h0 - agg1).sum(-1)
    sel1, n01 = topk_sel(score1, k1, None)
    xn1 = jnp.where(sel1[:, None], h0 * jnp.tanh(score1)[:, None], 0.0)
    x1 = readout(xn1, k1)

    NEG = jnp.float32(-1e9)
    nh = xn1.shape[1]
    ab1 = _mm_bias(xn1, jnp.stack([att1[:nh], att1[nh:]], axis=1),
                   jnp.zeros((2,), f32))
    aP1 = jnp.where(sel1, ab1[:, 0], NEG)
    bP1 = jnp.where(sel1, ab1[:, 1], NEG)
    zeros_e = jnp.zeros((src.shape[0],), f32)
    ex1, den1_p = _sc_att(aP1, bP1, src, dst, dst, aP1, zeros_e, zeros_e,
                          stage2=False)
    den1 = den1_p[0] + den1_p[1]
    r1 = 1.0 / jnp.maximum(den1, 1e-16)
    valid1 = ex1 > 0.0

    h2in = _mm_bias(xn1, W2, b2)
    degw2 = den1 * r1 + 1.0
    aggw2 = r1[:, None] * _seg_sum_rows(h2in, src, dst, ex1, n) + h2in
    h1 = jax.nn.relu(aggw2 / degw2[:, None])

    degp2 = den1 * r1
    aggp2 = r1[:, None] * _seg_sum_rows(h1, src, dst, ex1, n) \
        / jnp.maximum(degp2, 1e-9)[:, None]
    score2 = jnp.abs(h1 - aggp2).sum(-1)
    sel2, _ = topk_sel(score2, k2, sel1)
    xn2 = jnp.where(sel2[:, None], h1 * jnp.tanh(score2)[:, None], 0.0)
    x2 = readout(xn2, k2)

    ab2 = _mm_bias(xn2, jnp.stack([att2[:nh], att2[nh:]], axis=1),
                   jnp.zeros((2,), f32))
    aP2 = jnp.where(sel2, ab2[:, 0], NEG)
    bP2 = jnp.where(sel2, ab2[:, 1], NEG)
    v1f = valid1.astype(f32)
    s_art = aP2[n01] + bP2[n01]
    ex2_art = jnp.exp(jnp.maximum(s_art, 0.2 * s_art))
    ex2k, den2_p = _sc_att(aP2, bP2, src, dst, dst, r1, ex1, v1f, stage2=True)
    ex2 = jnp.where(valid1, ex2k, ex2_art)
    csum = (src.shape[0] - jnp.sum(v1f)) * ex2_art
    den2 = (den2_p[0] + den2_p[1]).at[n01].add(csum)
    r2 = 1.0 / jnp.maximum(den2, 1e-16)

    h3 = _mm_bias(xn2, W3, b3)
    degw3 = den2 * r2 + 1.0
    w3m = jnp.where(valid1, ex2, 0.0)
    agg3 = _seg_sum_rows(h3, src, dst, w3m, n).at[n01].add(csum * h3[n01])
    agg3 = r2[:, None] * agg3 + h3
    h2 = jnp.where(sel2[:, None], jax.nn.relu(agg3 / degw3[:, None]), 0.0)
    x3 = readout(h2, k2)

    xr = jax.nn.relu(x1) + jax.nn.relu(x2) + jax.nn.relu(x3)
    xr = jax.nn.relu(xr @ Wl1 + bl1)
    xr = jax.nn.relu(xr @ Wl2 + bl2)
    return jax.nn.log_softmax(xr @ Wl3 + bl3, axis=-1)


# speedup vs baseline: 19.8333x; 1.1359x over previous
import functools
import jax
import jax.numpy as jnp
from jax import lax
from jax.experimental import pallas as pl
from jax.experimental.pallas import tpu as pltpu
from jax.experimental.pallas import tpu_sc as plsc

_NC = 2
_NS = 16
_NW = _NC * _NS


def _mesh():
    return plsc.VectorSubcoreMesh(core_axis_name="c", subcore_axis_name="s")


@functools.partial(jax.jit, static_argnames=("weighted",))
def _sc_rows_agg(h, Sidx, Tidx, w, *, weighted):
    n, D = h.shape
    E = Sidx.shape[0]
    C = 128
    assert E % C == 0 and n % _NS == 0
    nchunk_total = E // C
    rounds = (nchunk_total + _NW - 1) // _NW
    assert n % 16 == 0
    nblk = n // 16
    blk_iters = (nblk + _NS - 1) // _NS
    nD8 = D // 16

    def body(h_hbm, s_hbm, t_hbm, w_hbm, out_hbm, sidx_v, tidx_v, w_v,
             rows_v, zbuf_v, accum_sh, sem):
        cid = lax.axis_index("c")
        sid = lax.axis_index("s")
        wid = cid * _NS + sid

        zero16 = jnp.zeros((16,), jnp.float32)
        for r in range(16):
            for j in range(nD8):
                zbuf_v[r, pl.ds(j * 16, 16)] = zero16

        def zcopy(i, _):
            b = sid + i * _NS

            @pl.when(b < nblk)
            def _do():
                pltpu.sync_copy(zbuf_v, accum_sh.at[pl.ds(b * 16, 16)])

            return _

        lax.fori_loop(0, blk_iters, zcopy, 0)
        plsc.subcore_barrier()

        def chunk(i, _):
            ck = wid + i * _NW

            @pl.when(ck < nchunk_total)
            def _do():
                base = ck * C
                pltpu.sync_copy(s_hbm.at[pl.ds(base, C)], sidx_v)
                pltpu.sync_copy(t_hbm.at[pl.ds(base, C)], tidx_v)
                pltpu.async_copy(h_hbm.at[sidx_v], rows_v, sem).wait()
                if weighted:
                    pltpu.sync_copy(w_hbm.at[pl.ds(base, C)], w_v)
                    for b in range(C // 16):
                        w16 = w_v[pl.ds(b * 16, 16)]
                        for l in range(16):
                            r = b * 16 + l
                            ws = w16[l]
                            for j in range(nD8):
                                sl = pl.ds(j * 16, 16)
                                rows_v[r, sl] = rows_v[r, sl] * ws
                pltpu.sync_copy(rows_v, accum_sh.at[tidx_v], add=True)

            return _

        lax.fori_loop(0, rounds, chunk, 0)
        plsc.subcore_barrier()

        def wb(i, _):
            b = sid + i * _NS

            @pl.when(b < nblk)
            def _do():
                sl = pl.ds(b * 16, 16)
                pltpu.sync_copy(accum_sh.at[sl], zbuf_v)
                pltpu.sync_copy(zbuf_v, out_hbm.at[cid].at[sl])

            return _

        lax.fori_loop(0, blk_iters, wb, 0)

    f = pl.kernel(
        body,
        out_type=jax.ShapeDtypeStruct((_NC, n, D), jnp.float32),
        mesh=_mesh(),
        scratch_types=[
            pltpu.VMEM((C,), jnp.int32),
            pltpu.VMEM((C,), jnp.int32),
            pltpu.VMEM((C,), jnp.float32),
            pltpu.VMEM((C, D), jnp.float32),
            pltpu.VMEM((16, D), jnp.float32),
            pltpu.VMEM_SHARED((n, D), jnp.float32),
            pltpu.SemaphoreType.DMA,
        ],
    )
    return f(h, Sidx, Tidx, w)


def _scalar_accum_helpers(n):
    base_len = (n // (16 * 128)) * 128
    last_len = n - 15 * base_len
    return base_len, last_len


@functools.partial(jax.jit, static_argnames=("stage2",))
def _sc_att(aP, bP, Sg, Tg, Tsc, rprev, exprev, v1, *, stage2):
    n = aP.shape[0]
    E = Sg.shape[0]
    per_w = E // _NW
    C = 80
    assert E % _NW == 0 and per_w % C == 0
    nchunks = per_w // C
    base_len, last_len = _scalar_accum_helpers(n)

    def body(a_hbm, b_hbm, sg_hbm, tg_hbm, tsc_hbm, r_hbm, exp_hbm, v1_hbm,
             ex_hbm, den_hbm, sgi_v, tgi_v, tsi_v, as_v, bs_v, rt_v, exi_v,
             v1_v, ex_v, sct_v, zbuf_v, accum_sh, sem):
        cid = lax.axis_index("c")
        sid = lax.axis_index("s")
        wid = cid * _NS + sid
        base_w = wid * per_w

        zero16 = jnp.zeros((16,), jnp.float32)
        for j in range(max(base_len, last_len) // 16):
            zbuf_v[pl.ds(j * 16, 16)] = zero16

        abase = sid * base_len

        @pl.when(sid < 15)
        def _z0():
            pltpu.sync_copy(zbuf_v.at[pl.ds(0, base_len)],
                            accum_sh.at[pl.ds(abase, base_len)])

        @pl.when(sid == 15)
        def _z1():
            pltpu.sync_copy(zbuf_v.at[pl.ds(0, last_len)],
                            accum_sh.at[pl.ds(15 * base_len, last_len)])

        plsc.subcore_barrier()

        def chunk(i, _):
            base = base_w + i * C
            pltpu.sync_copy(sg_hbm.at[pl.ds(base, C)], sgi_v)
            pltpu.sync_copy(tg_hbm.at[pl.ds(base, C)], tgi_v)
            pltpu.sync_copy(tsc_hbm.at[pl.ds(base, C)], tsi_v)
            pltpu.async_copy(a_hbm.at[sgi_v], as_v, sem).wait()
            pltpu.async_copy(b_hbm.at[tgi_v], bs_v, sem).wait()
            if stage2:
                pltpu.async_copy(r_hbm.at[tgi_v], rt_v, sem).wait()
                pltpu.sync_copy(exp_hbm.at[pl.ds(base, C)], exi_v)
                pltpu.sync_copy(v1_hbm.at[pl.ds(base, C)], v1_v)
            for k in range(C // 16):
                sl = pl.ds(k * 16, 16)
                s = as_v[sl] + bs_v[sl]
                logit = jnp.maximum(s, 0.2 * s)
                if stage2:
                    logit = logit + exi_v[sl] * rt_v[sl]
                else:
                    logit = logit + 1.0
                e = jnp.exp(logit)
                ex_v[sl] = e
                sct_v[sl] = e * v1_v[sl] if stage2 else e
            pltpu.sync_copy(ex_v, ex_hbm.at[pl.ds(base, C)])
            pltpu.sync_copy(sct_v, accum_sh.at[tsi_v], add=True)
            return _

        lax.fori_loop(0, nchunks, chunk, 0)
        plsc.subcore_barrier()

        @pl.when(sid < 15)
        def _w0():
            pltpu.sync_copy(accum_sh.at[pl.ds(abase, base_len)],
                            zbuf_v.at[pl.ds(0, base_len)])
            pltpu.sync_copy(zbuf_v.at[pl.ds(0, base_len)],
                            den_hbm.at[cid].at[pl.ds(abase, base_len)])

        @pl.when(sid == 15)
        def _w1():
            pltpu.sync_copy(accum_sh.at[pl.ds(15 * base_len, last_len)],
                            zbuf_v.at[pl.ds(0, last_len)])
            pltpu.sync_copy(zbuf_v.at[pl.ds(0, last_len)],
                            den_hbm.at[cid].at[pl.ds(15 * base_len, last_len)])

    zlen = 16 * ((max(*_scalar_accum_helpers(n)) + 15) // 16)
    f = pl.kernel(
        body,
        out_type=(jax.ShapeDtypeStruct((E,), jnp.float32),
                  jax.ShapeDtypeStruct((_NC, n), jnp.float32)),
        mesh=_mesh(),
        scratch_types=[
            pltpu.VMEM((C,), jnp.int32),
            pltpu.VMEM((C,), jnp.int32),
            pltpu.VMEM((C,), jnp.int32),
            pltpu.VMEM((C,), jnp.float32),
            pltpu.VMEM((C,), jnp.float32),
            pltpu.VMEM((C,), jnp.float32),
            pltpu.VMEM((C,), jnp.float32),
            pltpu.VMEM((C,), jnp.float32),
            pltpu.VMEM((C,), jnp.float32),
            pltpu.VMEM((C,), jnp.float32),
            pltpu.VMEM((zlen,), jnp.float32),
            pltpu.VMEM_SHARED((n,), jnp.float32),
            pltpu.SemaphoreType.DMA,
        ],
    )
    return f(aP, bP, Sg, Tg, Tsc, rprev, exprev, v1)


@functools.partial(jax.jit, static_argnames=("n",))
def _sc_count(Tidx, n):
    E = Tidx.shape[0]
    per_w = E // _NW
    C = 80
    assert E % _NW == 0 and per_w % C == 0
    nchunks = per_w // C
    base_len, last_len = _scalar_accum_helpers(n)

    def body(t_hbm, out_hbm, tidx_v, ones_v, zbuf_v, accum_sh):
        cid = lax.axis_index("c")
        sid = lax.axis_index("s")
        wid = cid * _NS + sid
        base_w = wid * per_w

        zero16 = jnp.zeros((16,), jnp.float32)
        one16 = jnp.ones((16,), jnp.float32)
        for j in range(C // 16):
            ones_v[pl.ds(j * 16, 16)] = one16
        for j in range(max(base_len, last_len) // 16):
            zbuf_v[pl.ds(j * 16, 16)] = zero16

        abase = sid * base_len

        @pl.when(sid < 15)
        def _z0():
            pltpu.sync_copy(zbuf_v.at[pl.ds(0, base_len)],
                            accum_sh.at[pl.ds(abase, base_len)])

        @pl.when(sid == 15)
        def _z1():
            pltpu.sync_copy(zbuf_v.at[pl.ds(0, last_len)],
                            accum_sh.at[pl.ds(15 * base_len, last_len)])

        plsc.subcore_barrier()

        def chunk(i, _):
            base = base_w + i * C
            pltpu.sync_copy(t_hbm.at[pl.ds(base, C)], tidx_v)
            pltpu.sync_copy(ones_v, accum_sh.at[tidx_v], add=True)
            return _

        lax.fori_loop(0, nchunks, chunk, 0)
        plsc.subcore_barrier()

        @pl.when(sid < 15)
        def _w0():
            pltpu.sync_copy(accum_sh.at[pl.ds(abase, base_len)],
                            zbuf_v.at[pl.ds(0, base_len)])
            pltpu.sync_copy(zbuf_v.at[pl.ds(0, base_len)],
                            out_hbm.at[cid].at[pl.ds(abase, base_len)])

        @pl.when(sid == 15)
        def _w1():
            pltpu.sync_copy(accum_sh.at[pl.ds(15 * base_len, last_len)],
                            zbuf_v.at[pl.ds(0, last_len)])
            pltpu.sync_copy(zbuf_v.at[pl.ds(0, last_len)],
                            out_hbm.at[cid].at[pl.ds(15 * base_len, last_len)])

    zlen = 16 * ((max(base_len, last_len) + 15) // 16)
    f = pl.kernel(
        body,
        out_type=jax.ShapeDtypeStruct((_NC, n), jnp.float32),
        mesh=_mesh(),
        scratch_types=[
            pltpu.VMEM((C,), jnp.int32),
            pltpu.VMEM((C,), jnp.float32),
            pltpu.VMEM((zlen,), jnp.float32),
            pltpu.VMEM_SHARED((n,), jnp.float32),
        ],
    )
    return f(Tidx)


@functools.partial(jax.jit, static_argnames=("k",))
def _tc_topk_threshold(msc_pad, k):
    R, L = msc_pad.shape

    def body(v_ref, tau_ref, idx_ref):
        v = v_ref[...]
        lo = jnp.min(v)
        hi = jnp.max(v)

        def it(_, carry):
            lo, hi = carry
            mid = 0.5 * (lo + hi)
            cnt = jnp.sum((v >= mid).astype(jnp.float32))
            pred = cnt >= k
            return jnp.where(pred, mid, lo), jnp.where(pred, hi, mid)

        lo, hi = lax.fori_loop(0, 45, it, (lo, hi))
        tau_ref[...] = jnp.reshape(lo, (1, 1))
        mx = jnp.max(v)
        row = lax.broadcasted_iota(jnp.int32, (R, L), 0)
        col = lax.broadcasted_iota(jnp.int32, (R, L), 1)
        flat = row * L + col
        idx_ref[...] = jnp.reshape(jnp.min(jnp.where(v == mx, flat, R * L)),
                                   (1, 1))

    return pl.pallas_call(
        body,
        out_shape=(jax.ShapeDtypeStruct((1, 1), jnp.float32),
                   jax.ShapeDtypeStruct((1, 1), jnp.int32)),
    )(msc_pad)


def _mm_bias(X, W, b):
    n, d = X.shape
    m = W.shape[1]
    blk = 1000
    assert n % blk == 0

    def body(x_ref, w_ref, b_ref, o_ref):
        o_ref[...] = jnp.dot(x_ref[...], w_ref[...],
                             preferred_element_type=jnp.float32) + b_ref[...]

    return pl.pallas_call(
        body,
        grid=(n // blk,),
        in_specs=[
            pl.BlockSpec((blk, d), lambda i: (i, 0)),
            pl.BlockSpec((d, m), lambda i: (0, 0)),
            pl.BlockSpec((1, m), lambda i: (0, 0)),
        ],
        out_specs=pl.BlockSpec((blk, m), lambda i: (i, 0)),
        out_shape=jax.ShapeDtypeStruct((n, m), jnp.float32),
    )(X, W, b.reshape(1, m))


def _seg_sum_rows(h, Sidx, Tidx, w, n):
    weighted = w is not None
    if not weighted:
        w = jnp.zeros((1,), h.dtype)
        w = jnp.broadcast_to(w, (Sidx.shape[0],))
    p = _sc_rows_agg(h, Sidx, Tidx, w, weighted=weighted)
    return p[0] + p[1]


def kernel(x, edge_index, batch, W1, b1, W2, b2, W3, b3, att1, att2,
           Wl1, bl1, Wl2, bl2, Wl3, bl3):
    n = x.shape[0]
    k1 = n // 2
    k2 = k1 // 2
    src = edge_index[0]
    dst = edge_index[1]
    f32 = x.dtype

    def readout(xn, k):
        return jnp.concatenate([jnp.max(xn, axis=0), jnp.sum(xn, axis=0) / k])[None, :]

    def topk_sel(score, k, sel_prev):
        msc = score if sel_prev is None else jnp.where(sel_prev, score, -1.0)
        npad = ((n + 127) // 128) * 128
        mp = jnp.pad(msc, (0, npad - n), constant_values=-3.0)
        tau, n0 = _tc_topk_threshold(mp.reshape(npad // 128, 128), k)
        return msc >= tau[0, 0], n0[0, 0]

    cnt_p = _sc_count(dst, n)
    cnt = cnt_p[0] + cnt_p[1]
    dinv = jax.lax.rsqrt(jnp.maximum(cnt + 1.0, 1.0))
    g = dinv[:, None] * _mm_bias(x, W1, b1)
    aggA = _seg_sum_rows(g, src, dst, None, n)
    h0 = jax.nn.relu(dinv[:, None] * (aggA + g))

    agg1 = _seg_sum_rows(h0, src, dst, None, n) \
        / jnp.maximum(cnt, 1e-9)[:, None]
    score1 = jnp.abs(h0 - agg1).sum(-1)
    sel1, n01 = topk_sel(score1, k1, None)
    xn1 = jnp.where(sel1[:, None], h0 * jnp.tanh(score1)[:, None], 0.0)
    x1 = readout(xn1, k1)

    NEG = jnp.float32(-1e9)
    nh = xn1.shape[1]
    ab1 = _mm_bias(xn1, jnp.stack([att1[:nh], att1[nh:]], axis=1),
                   jnp.zeros((2,), f32))
    aP1 = jnp.where(sel1, ab1[:, 0], NEG)
    bP1 = jnp.where(sel1, ab1[:, 1], NEG)
    zeros_e = jnp.zeros((src.shape[0],), f32)
    ex1, den1_p = _sc_att(aP1, bP1, src, dst, dst, aP1, zeros_e, zeros_e,
                          stage2=False)
    den1 = den1_p[0] + den1_p[1]
    r1 = 1.0 / jnp.maximum(den1, 1e-16)
    valid1 = ex1 > 0.0

    h2in = _mm_bias(xn1, W2, b2)
    degw2 = den1 * r1 + 1.0
    aggw2 = r1[:, None] * _seg_sum_rows(h2in, src, dst, ex1, n) + h2in
    h1 = jax.nn.relu(aggw2 / degw2[:, None])

    degp2 = den1 * r1
    aggp2 = r1[:, None] * _seg_sum_rows(h1, src, dst, ex1, n) \
        / jnp.maximum(degp2, 1e-9)[:, None]
    score2 = jnp.abs(h1 - aggp2).sum(-1)
    sel2, _ = topk_sel(score2, k2, sel1)
    xn2 = jnp.where(sel2[:, None], h1 * jnp.tanh(score2)[:, None], 0.0)
    x2 = readout(xn2, k2)

    ab2 = _mm_bias(xn2, jnp.stack([att2[:nh], att2[nh:]], axis=1),
                   jnp.zeros((2,), f32))
    aP2 = jnp.where(sel2, ab2[:, 0], NEG)
    bP2 = jnp.where(sel2, ab2[:, 1], NEG)
    v1f = valid1.astype(f32)
    s_art = aP2[n01] + bP2[n01]
    ex2_art = jnp.exp(jnp.maximum(s_art, 0.2 * s_art))
    ex2k, den2_p = _sc_att(aP2, bP2, src, dst, dst, r1, ex1, v1f, stage2=True)
    ex2 = jnp.where(valid1, ex2k, ex2_art)
    csum = (src.shape[0] - jnp.sum(v1f)) * ex2_art
    den2 = (den2_p[0] + den2_p[1]).at[n01].add(csum)
    r2 = 1.0 / jnp.maximum(den2, 1e-16)

    h3 = _mm_bias(xn2, W3, b3)
    degw3 = den2 * r2 + 1.0
    w3m = jnp.where(valid1, ex2, 0.0)
    agg3 = _seg_sum_rows(h3, src, dst, w3m, n).at[n01].add(csum * h3[n01])
    agg3 = r2[:, None] * agg3 + h3
    h2 = jnp.where(sel2[:, None], jax.nn.relu(agg3 / degw3[:, None]), 0.0)
    x3 = readout(h2, k2)

    xr = jax.nn.relu(x1) + jax.nn.relu(x2) + jax.nn.relu(x3)
    xr = jax.nn.relu(xr @ Wl1 + bl1)
    xr = jax.nn.relu(xr @ Wl2 + bl2)
    return jax.nn.log_softmax(xr @ Wl3 + bl3, axis=-1)
